# baseline (speedup 1.0000x reference)
import functools

import jax
import jax.numpy as jnp
from jax import lax
from jax.experimental import pallas as pl
from jax.experimental.pallas import tpu as pltpu
from jax.experimental.pallas import tpu_sc as plsc

N = 10000
E = 160000
NNZ = 320000
G = 64
F = 128

NC = 2
NS = 16
NTILES = NC * NS
CHUNK = 128
CEXT = 144
CH2 = 64

_mesh = lambda: plsc.VectorSubcoreMesh(core_axis_name="c", subcore_axis_name="s")


def _sc_edge_gather(x, edge_index, batch):
    nchunks = E // CHUNK
    npt = -(-nchunks // NTILES)
    np3 = -(-npt // 3)

    buf = lambda: [pltpu.VMEM((CHUNK,), jnp.int32),
                   pltpu.VMEM((CHUNK,), jnp.int32),
                   pltpu.VMEM((CHUNK, F), jnp.float32),
                   pltpu.VMEM((CHUNK, F), jnp.float32),
                   pltpu.VMEM((CHUNK,), jnp.int32),
                   pltpu.SemaphoreType.DMA,
                   pltpu.SemaphoreType.DMA,
                   pltpu.SemaphoreType.DMA]

    @functools.partial(
        pl.kernel,
        mesh=_mesh(),
        out_type=[
            jax.ShapeDtypeStruct((E, F), jnp.float32),
            jax.ShapeDtypeStruct((E, F), jnp.float32),
            jax.ShapeDtypeStruct((E,), jnp.int32),
        ],
        scratch_types=buf() + buf() + buf(),
    )
    def k(x_hbm, eidx_hbm, batch_hbm, gxc_hbm, gxr_hbm, bcol_hbm,
          *rest):
        wid = lax.axis_index("s") * NC + lax.axis_index("c")
        bufs = (rest[0:8], rest[8:16], rest[16:24])

        def issue(i, b):
            ci, ri, rc, rr, bv, s1, s2, s3 = bufs[b]
            ch = wid + i * NTILES

            @pl.when(ch < nchunks)
            def _():
                base = ch * CHUNK
                pltpu.sync_copy(eidx_hbm.at[0, pl.ds(base, CHUNK)], ci)
                pltpu.sync_copy(eidx_hbm.at[1, pl.ds(base, CHUNK)], ri)
                pltpu.async_copy(x_hbm.at[ci], rc, s1)
                pltpu.async_copy(x_hbm.at[ri], rr, s2)
                pltpu.async_copy(batch_hbm.at[ci], bv, s3)

        def drain(i, b):
            ci, ri, rc, rr, bv, s1, s2, s3 = bufs[b]
            ch = wid + i * NTILES

            @pl.when(ch < nchunks)
            def _():
                base = ch * CHUNK
                pltpu.make_async_copy(x_hbm.at[ci], rc, s1).wait()
                pltpu.make_async_copy(x_hbm.at[ri], rr, s2).wait()
                pltpu.make_async_copy(batch_hbm.at[ci], bv, s3).wait()
                pltpu.sync_copy(rc, gxc_hbm.at[pl.ds(base, CHUNK)])
                pltpu.sync_copy(rr, gxr_hbm.at[pl.ds(base, CHUNK)])
                pltpu.sync_copy(bv, bcol_hbm.at[pl.ds(base, CHUNK)])

        issue(0, 0)
        issue(1, 1)

        def body(i3, carry):
            i = 3 * i3
            issue(i + 2, 2)
            drain(i, 0)
            issue(i + 3, 0)
            drain(i + 1, 1)
            issue(i + 4, 1)
            drain(i + 2, 2)
            return carry

        lax.fori_loop(0, np3, body, 0)

    return k(x, edge_index, batch)


NP = 10240
NCHUNKS = NNZ // CHUNK


def _sc_hconv_pass(table, hyper_edge, gdim, sdim, stacked,
                   scalar_mode=None, nw=None):
    stride = NS if stacked else NTILES
    npt = -(-NCHUNKS // stride)
    np2 = -(-npt // 2)
    rpt = NP // NS
    nsub = CHUNK // 16

    out_type = [jax.ShapeDtypeStruct((NC, NP, F), jnp.float32)]
    if scalar_mode:
        out_type.append(jax.ShapeDtypeStruct((NP,), jnp.float32))

    nb = 7 if scalar_mode == "dn" else 5
    buf = lambda: ([pltpu.VMEM((CHUNK,), jnp.int32),
                    pltpu.VMEM((CHUNK,), jnp.int32),
                    pltpu.VMEM((CHUNK,), jnp.int32),
                    pltpu.VMEM((CHUNK, F), jnp.float32),
                    pltpu.SemaphoreType.DMA]
                   + ([pltpu.VMEM((CHUNK,), jnp.float32),
                       pltpu.SemaphoreType.DMA]
                      if scalar_mode == "dn" else []))
    scratch = buf() + buf() + [
        pltpu.VMEM_SHARED((NP, F), jnp.float32),
    ]
    if scalar_mode:
        scratch.append(pltpu.VMEM_SHARED((NP,), jnp.float32))
    if scalar_mode == "cnt":
        scratch.append(pltpu.VMEM((CHUNK,), jnp.float32))

    @functools.partial(pl.kernel, mesh=_mesh(), out_type=out_type,
                       scratch_types=scratch)
    def k(tab_hbm, he_hbm, zero_hbm, zero1_hbm, ones_hbm, nw_hbm,
          *rest):
        if scalar_mode:
            acc_out, sc_out = rest[0], rest[1]
            rest = rest[2:]
        else:
            acc_out = rest[0]
            rest = rest[1:]
        bufs = (rest[0:nb], rest[nb:2 * nb])
        rest = rest[2 * nb:]
        acc_sh = rest[0]
        sacc_sh = rest[1] if scalar_mode else None
        ones_v = rest[2] if scalar_mode == "cnt" else None
        cid = lax.axis_index("c")
        sid = lax.axis_index("s")
        wid = sid * NC + cid
        goff = cid * N
        base0 = sid if stacked else wid

        tb = sid * rpt
        pltpu.sync_copy(zero_hbm.at[pl.ds(tb, rpt)], acc_sh.at[pl.ds(tb, rpt)])
        if scalar_mode == "cnt":
            pltpu.sync_copy(ones_hbm, ones_v)
        if scalar_mode:
            @pl.when((cid == 0) & (sid == 0))
            def _():
                pltpu.sync_copy(zero1_hbm, sacc_sh)
        plsc.subcore_barrier()

        def parts(b):
            t = bufs[b]
            w, sem2 = (t[5], t[6]) if nb == 7 else (None, None)
            return t[0], t[1], t[2], t[3], t[4], w, sem2

        def gref(b):
            gio, gim, si, rows, sem, w, sem2 = parts(b)
            return gim if stacked else gio

        def issue(i, b):
            gio, gim, si, rows, sem, w, sem2 = parts(b)
            ch = base0 + i * stride

            @pl.when(ch < NCHUNKS)
            def _():
                base = ch * CHUNK
                pltpu.sync_copy(he_hbm.at[gdim, pl.ds(base, CHUNK)], gio)
                pltpu.sync_copy(he_hbm.at[sdim, pl.ds(base, CHUNK)], si)
                if scalar_mode == "dn":
                    @pl.when(cid == 0)
                    def _():
                        pltpu.async_copy(nw_hbm.at[gio], w, sem2)
                if stacked:
                    for j in range(nsub):
                        sl = pl.ds(j * 16, 16)
                        gim[sl] = gio[sl] + goff
                pltpu.async_copy(tab_hbm.at[gref(b)], rows, sem)

        def drain(i, b):
            gio, gim, si, rows, sem, w, sem2 = parts(b)
            ch = base0 + i * stride

            @pl.when(ch < NCHUNKS)
            def _():
                pltpu.make_async_copy(tab_hbm.at[gref(b)], rows, sem).wait()
                pltpu.sync_copy(rows, acc_sh.at[si], add=True)
                if scalar_mode == "cnt":
                    @pl.when(cid == 0)
                    def _():
                        pltpu.sync_copy(ones_v, sacc_sh.at[si], add=True)
                elif scalar_mode == "dn":
                    @pl.when(cid == 0)
                    def _():
                        pltpu.make_async_copy(nw_hbm.at[gio], w, sem2).wait()
                        pltpu.sync_copy(w, sacc_sh.at[si], add=True)

        issue(0, 0)

        def body(i2, carry):
            i = 2 * i2
            issue(i + 1, 1)
            drain(i, 0)
            issue(i + 2, 0)
            drain(i + 1, 1)
            return carry

        lax.fori_loop(0, np2, body, 0)
        plsc.subcore_barrier()
        pltpu.sync_copy(acc_sh.at[pl.ds(tb, rpt)],
                        acc_out.at[cid, pl.ds(tb, rpt)])
        if scalar_mode:
            @pl.when(cid == 0)
            def _():
                pltpu.sync_copy(sacc_sh.at[pl.ds(tb, rpt)],
                                sc_out.at[pl.ds(tb, rpt)])

    zeros = jnp.zeros((NP, F), jnp.float32)
    zeros1 = jnp.zeros((NP,), jnp.float32)
    ones = jnp.ones((CHUNK,), jnp.float32)
    if nw is None:
        nw = jnp.zeros((N,), jnp.float32)
    res = k(table, hyper_edge, zeros, zeros1, ones, nw)
    return res if scalar_mode else res[0]


NBLK = 2000
NNB = N // NBLK

_full = lambda shape: pl.BlockSpec(shape, lambda i: (0,) * len(shape))
_nrow = lambda w: pl.BlockSpec((NBLK, w), lambda i: (i, 0))


def _node_body(gemb_tab_ref, whl1a_ref, whl1b_ref, bhl1_ref, whl2_ref,
               bhl2_ref, x_ref, batch_ref, nw_ref):
    b = batch_ref[0, 0]
    onehot = (b[:, None] == lax.broadcasted_iota(jnp.int32, (1, G), 1)
              ).astype(jnp.float32)
    proto = jnp.dot(onehot, gemb_tab_ref[...],
                    preferred_element_type=jnp.float32)
    h = jnp.dot(x_ref[...], whl1a_ref[...], preferred_element_type=jnp.float32)
    h += jnp.dot(proto, whl1b_ref[...], preferred_element_type=jnp.float32)
    h = jnp.maximum(h + bhl1_ref[...], 0.0)
    nw = jnp.dot(h, whl2_ref[...], preferred_element_type=jnp.float32) \
        + bhl2_ref[...]
    nw_ref[...] = jax.nn.sigmoid(nw)


def _node_stage(x, graph_emb, batch, Whl1, bhl1, Whl2, bhl2):
    return pl.pallas_call(
        _node_body,
        grid=(NNB,),
        in_specs=[
            _full((G, F)), _full((F, F)), _full((F, F)), _full((1, F)),
            _full((F, 1)), _full((1, 1)),
            _nrow(F),
            pl.BlockSpec((1, 1, NBLK), lambda i: (i, 0, 0)),
        ],
        out_specs=_nrow(1),
        out_shape=jax.ShapeDtypeStruct((N, 1), jnp.float32),
    )(graph_emb, Whl1[:F], Whl1[F:], bhl1.reshape(1, -1), Whl2,
      bhl2.reshape(1, -1), x, batch.reshape(NNB, 1, NBLK))


_stk = pl.BlockSpec((2, NBLK, F), lambda i: (0, i, 0))
_scal = pl.BlockSpec((NBLK, 1), lambda i: (i, 0))


def _prep1_body(wa_ref, wb_ref, b_ref, gxc_ref, gxr_ref, out_ref):
    Xl = jnp.dot(gxc_ref[...], wa_ref[...], preferred_element_type=jnp.float32)
    Xl += jnp.dot(gxr_ref[...], wb_ref[...], preferred_element_type=jnp.float32)
    Xl += b_ref[...]
    out_ref[...] = jnp.stack([Xl[:, :F], Xl[:, F:]], axis=0)


def _prep1(gxc, gxr, Whc1, bhc1):
    return pl.pallas_call(
        _prep1_body,
        grid=(NNB,),
        in_specs=[_full((F, 2 * F)), _full((F, 2 * F)), _full((1, 2 * F)),
                  _nrow(F), _nrow(F)],
        out_specs=_stk,
        out_shape=jax.ShapeDtypeStruct((2, N, F), jnp.float32),
    )(Whc1[:F], Whc1[F:], bhc1.reshape(1, -1), gxc, gxr)


def _prep2_body(s1_ref, cntp_ref, nw_ref, out_ref, scale_ref):
    cnt = cntp_ref[...][:, 0]
    nw = nw_ref[...][:, 0]
    s = jnp.where(cnt > 0, nw / cnt, 0.0)
    out_ref[...] = s[None, :, None] * s1_ref[...]
    scale_ref[...] = s[:, None]


def _prep2(S1, cntp, nw):
    return pl.pallas_call(
        _prep2_body,
        grid=(NNB,),
        in_specs=[_stk, _scal, _nrow(1)],
        out_specs=[_stk, _nrow(1)],
        out_shape=[jax.ShapeDtypeStruct((2, N, F), jnp.float32),
                   jax.ShapeDtypeStruct((N, 1), jnp.float32)],
    )(S1, cntp, nw)


def _prep3_body(whc2a_ref, whc2b_ref, b_ref, t1_ref, dnp_ref,
                out_ref, dinv_ref):
    dn = dnp_ref[...][:, 0]
    dinv = jnp.where(dn > 0, 1.0 / dn, 0.0)
    era = jax.nn.sigmoid(dinv[:, None] * t1_ref[0])
    erb = jax.nn.sigmoid(dinv[:, None] * t1_ref[1])
    Xl2 = jnp.dot(era, whc2a_ref[...], preferred_element_type=jnp.float32)
    Xl2 += jnp.dot(erb, whc2b_ref[...], preferred_element_type=jnp.float32)
    out_ref[...] = Xl2 + b_ref[...]
    dinv_ref[...] = dinv[:, None]


def _prep3(T1, dnp, Whc2, bhc2):
    return pl.pallas_call(
        _prep3_body,
        grid=(NNB,),
        in_specs=[_full((F, F)), _full((F, F)), _full((1, F)), _stk, _scal],
        out_specs=[_nrow(F), _nrow(1)],
        out_shape=[jax.ShapeDtypeStruct((N, F), jnp.float32),
                   jax.ShapeDtypeStruct((N, 1), jnp.float32)],
    )(Whc2[:F], Whc2[F:], bhc2.reshape(1, -1), T1, dnp)


def _prep4_body(s2_ref, scale_ref, out_ref):
    out_ref[...] = scale_ref[...] * (s2_ref[0] + s2_ref[1])


def _prep4(S2, scale):
    return pl.pallas_call(
        _prep4_body,
        grid=(NNB,),
        in_specs=[_stk, _nrow(1)],
        out_specs=_nrow(F),
        out_shape=jax.ShapeDtypeStruct((N, F), jnp.float32),
    )(S2, scale)


def _prep5_body(t2_ref, dinv_ref, out_ref):
    out_ref[...] = jax.nn.sigmoid(dinv_ref[...] * (t2_ref[0] + t2_ref[1]))


def _prep5(T2, dinv):
    return pl.pallas_call(
        _prep5_body,
        grid=(NNB,),
        in_specs=[_stk, _nrow(1)],
        out_specs=_nrow(F),
        out_shape=jax.ShapeDtypeStruct((N, F), jnp.float32),
    )(T2, dinv)


EBLK = 2000
NEB = E // EBLK
NSH = N // EBLK


def _edge_heavy_body(gemb_tab_ref, wl1a_ref, wl1b_ref, wl1c_ref, bl1_ref,
                     wl2_ref, bl2_ref, gxc_ref, gxr_ref, bcol_ref, out_ref):
    bcol = bcol_ref[0, 0]
    onehot = (bcol[:, None] == lax.broadcasted_iota(jnp.int32, (1, G), 1)
              ).astype(jnp.bfloat16)
    gemb = jnp.dot(onehot, gemb_tab_ref[...],
                   preferred_element_type=jnp.float32)
    h1 = jnp.dot(gxc_ref[...].astype(jnp.bfloat16), wl1a_ref[...],
                 preferred_element_type=jnp.float32)
    h1 += jnp.dot(gxr_ref[...].astype(jnp.bfloat16), wl1b_ref[...],
                  preferred_element_type=jnp.float32)
    h1 += jnp.dot(gemb.astype(jnp.bfloat16), wl1c_ref[...],
                  preferred_element_type=jnp.float32)
    h1 = jnp.maximum(h1 + bl1_ref[...], 0.0)
    out_ref[...] = jnp.maximum(
        jnp.dot(h1.astype(jnp.bfloat16), wl2_ref[...],
                preferred_element_type=jnp.float32)
        + bl2_ref[...], 0.0)


def _edge_heavy(gxc, gxr, bcol, graph_emb, Wl1, bl1, Wl2, bl2):
    return pl.pallas_call(
        _edge_heavy_body,
        grid=(NEB,),
        in_specs=[
            _full((G, F)),
            _full((F, 4 * F)), _full((F, 4 * F)), _full((F, 4 * F)),
            _full((1, 4 * F)),
            _full((4 * F, F)), _full((1, F)),
            pl.BlockSpec((EBLK, F), lambda i: (i, 0)),
            pl.BlockSpec((EBLK, F), lambda i: (i, 0)),
            pl.BlockSpec((1, 1, EBLK), lambda i: (i, 0, 0)),
        ],
        out_specs=pl.BlockSpec((EBLK, F), lambda i: (i, 0)),
        out_shape=jax.ShapeDtypeStruct((E, F), jnp.float32),
    )(graph_emb.astype(jnp.bfloat16), Wl1[:F].astype(jnp.bfloat16),
      Wl1[F:2 * F].astype(jnp.bfloat16), Wl1[2 * F:].astype(jnp.bfloat16),
      bl1.reshape(1, -1), Wl2.astype(jnp.bfloat16), bl2.reshape(1, -1),
      gxc, gxr, bcol.reshape(NEB, 1, EBLK))


def _edge_light_body(gemb_tab_ref, wc1a_ref, wc1b_ref, bc1_ref,
                     wc2_ref, bc2_ref, attn_ref,
                     xij2_ref, bcol_ref, t2_ref, dinv_ref, out_ref):
    pid = pl.program_id(0)
    bcol = bcol_ref[0, 0]
    onehot = (bcol[:, None] == lax.broadcasted_iota(jnp.int32, (1, G), 1)
              ).astype(jnp.bfloat16)
    gemb = jnp.dot(onehot, gemb_tab_ref[...],
                   preferred_element_type=jnp.float32)
    sh_n = jax.nn.sigmoid(dinv_ref[...] * (t2_ref[0] + t2_ref[1]))
    sh = jnp.where(pid < NSH, sh_n, 0.5)
    s = attn_ref[0, 0] * xij2_ref[...] + attn_ref[0, 1] * sh
    z = jnp.dot(s.astype(jnp.bfloat16), wc1a_ref[...],
                preferred_element_type=jnp.float32)
    z += jnp.dot(gemb.astype(jnp.bfloat16), wc1b_ref[...],
                 preferred_element_type=jnp.float32)
    z = jnp.maximum(z + bc1_ref[...], 0.0)
    o = jnp.dot(z.astype(jnp.bfloat16), wc2_ref[...],
                preferred_element_type=jnp.float32) + bc2_ref[...]
    out_ref[...] = jax.nn.sigmoid(o)


def _edge_light(xij2, bcol, T2, dinv, graph_emb, Wc1, bc1, Wc2, bc2, attn):
    clamp = lambda i: jnp.minimum(i, NSH - 1)
    return pl.pallas_call(
        _edge_light_body,
        grid=(NEB,),
        in_specs=[
            _full((G, F)),
            _full((F, F)), _full((F, F)), _full((1, F)),
            _full((F, 1)), _full((1, 1)),
            _full((1, 2)),
            pl.BlockSpec((EBLK, F), lambda i: (i, 0)),
            pl.BlockSpec((1, 1, EBLK), lambda i: (i, 0, 0)),
            pl.BlockSpec((2, EBLK, F), lambda i: (0, clamp(i), 0)),
            pl.BlockSpec((EBLK, 1), lambda i: (clamp(i), 0)),
        ],
        out_specs=pl.BlockSpec((EBLK, 1), lambda i: (i, 0)),
        out_shape=jax.ShapeDtypeStruct((E, 1), jnp.float32),
    )(graph_emb.astype(jnp.bfloat16), Wc1[:F].astype(jnp.bfloat16),
      Wc1[F:].astype(jnp.bfloat16), bc1.reshape(1, -1),
      Wc2.astype(jnp.bfloat16), bc2.reshape(1, -1), attn.reshape(1, 2),
      xij2, bcol.reshape(NEB, 1, EBLK), T2, dinv)


def kernel(x, graph_emb, edge_index, edge_type, batch, hyper_edge, attn,
           Whl1, bhl1, Whl2, bhl2, Whc1, bhc1, Whc2, bhc2,
           Wl1, bl1, Wl2, bl2, Wc1, bc1, Wc2, bc2):
    nw = _node_stage(x, graph_emb, batch, Whl1, bhl1, Whl2, bhl2)

    gxc, gxr, bcol = _sc_edge_gather(x, edge_index, batch)

    xij2 = _edge_heavy(gxc, gxr, bcol, graph_emb, Wl1, bl1, Wl2, bl2)

    Xl1e = _prep1(gxc, gxr, Whc1, bhc1)
    S1, cntp = _sc_hconv_pass(Xl1e.reshape(2 * N, F), hyper_edge, 0, 1,
                              stacked=True, scalar_mode="cnt")
    a1e, scale = _prep2(S1, cntp.reshape(NP, 1), nw)
    T1, dnp = _sc_hconv_pass(a1e.reshape(2 * N, F), hyper_edge, 1, 0,
                             stacked=True, scalar_mode="dn", nw=nw[:, 0])
    Xl2, dinv = _prep3(T1, dnp.reshape(NP, 1), Whc2, bhc2)
    S2 = _sc_hconv_pass(Xl2, hyper_edge, 0, 1, stacked=False)
    a2 = _prep4(S2, scale)
    T2 = _sc_hconv_pass(a2, hyper_edge, 1, 0, stacked=False)

    sij = _edge_light(xij2, bcol, T2, dinv, graph_emb,
                      Wc1, bc1, Wc2, bc2, attn)
    return (edge_index, edge_type, sij)


# speedup vs baseline: 1.0682x; 1.0682x over previous
import functools

import jax
import jax.numpy as jnp
from jax import lax
from jax.experimental import pallas as pl
from jax.experimental.pallas import tpu as pltpu
from jax.experimental.pallas import tpu_sc as plsc

N = 10000
E = 160000
NNZ = 320000
G = 64
F = 128

NC = 2
NS = 16
NTILES = NC * NS
CHUNK = 128
CEXT = 144
CH2 = 64

_mesh = lambda: plsc.VectorSubcoreMesh(core_axis_name="c", subcore_axis_name="s")


def _sc_edge_gather(x, edge_index, batch):
    nchunks = E // CHUNK
    npt = -(-nchunks // NTILES)
    np3 = -(-npt // 3)

    buf = lambda: [pltpu.VMEM((CHUNK,), jnp.int32),
                   pltpu.VMEM((CHUNK,), jnp.int32),
                   pltpu.VMEM((CHUNK, F), jnp.float32),
                   pltpu.VMEM((CHUNK, F), jnp.float32),
                   pltpu.VMEM((CHUNK,), jnp.int32),
                   pltpu.SemaphoreType.DMA,
                   pltpu.SemaphoreType.DMA,
                   pltpu.SemaphoreType.DMA]

    @functools.partial(
        pl.kernel,
        mesh=_mesh(),
        out_type=[
            jax.ShapeDtypeStruct((E, F), jnp.float32),
            jax.ShapeDtypeStruct((E, F), jnp.float32),
            jax.ShapeDtypeStruct((E,), jnp.int32),
        ],
        scratch_types=buf() + buf() + buf(),
    )
    def k(x_hbm, eidx_hbm, batch_hbm, gxc_hbm, gxr_hbm, bcol_hbm,
          *rest):
        wid = lax.axis_index("s") * NC + lax.axis_index("c")
        bufs = (rest[0:8], rest[8:16], rest[16:24])

        def issue(i, b):
            ci, ri, rc, rr, bv, s1, s2, s3 = bufs[b]
            ch = wid + i * NTILES

            @pl.when(ch < nchunks)
            def _():
                base = ch * CHUNK
                pltpu.sync_copy(eidx_hbm.at[0, pl.ds(base, CHUNK)], ci)
                pltpu.sync_copy(eidx_hbm.at[1, pl.ds(base, CHUNK)], ri)
                pltpu.async_copy(x_hbm.at[ci], rc, s1)
                pltpu.async_copy(x_hbm.at[ri], rr, s2)
                pltpu.async_copy(batch_hbm.at[ci], bv, s3)

        def drain(i, b):
            ci, ri, rc, rr, bv, s1, s2, s3 = bufs[b]
            ch = wid + i * NTILES

            @pl.when(ch < nchunks)
            def _():
                base = ch * CHUNK
                pltpu.make_async_copy(x_hbm.at[ci], rc, s1).wait()
                pltpu.make_async_copy(x_hbm.at[ri], rr, s2).wait()
                pltpu.make_async_copy(batch_hbm.at[ci], bv, s3).wait()
                pltpu.sync_copy(rc, gxc_hbm.at[pl.ds(base, CHUNK)])
                pltpu.sync_copy(rr, gxr_hbm.at[pl.ds(base, CHUNK)])
                pltpu.sync_copy(bv, bcol_hbm.at[pl.ds(base, CHUNK)])

        issue(0, 0)
        issue(1, 1)

        def body(i3, carry):
            i = 3 * i3
            issue(i + 2, 2)
            drain(i, 0)
            issue(i + 3, 0)
            drain(i + 1, 1)
            issue(i + 4, 1)
            drain(i + 2, 2)
            return carry

        lax.fori_loop(0, np3, body, 0)

    return k(x, edge_index, batch)


NP = 10240
NCHUNKS = NNZ // CHUNK


def _sc_hconv_pass(table, hyper_edge, gdim, sdim, stacked,
                   scalar_mode=None, nw=None, sched_dep=None):
    stride = NS if stacked else NTILES
    npt = -(-NCHUNKS // stride)
    np2 = -(-npt // 2)
    rpt = NP // NS
    nsub = CHUNK // 16

    out_type = [jax.ShapeDtypeStruct((NC, NP, F), jnp.float32)]
    if scalar_mode:
        out_type.append(jax.ShapeDtypeStruct((NP,), jnp.float32))

    nb = 7 if scalar_mode == "dn" else 5
    buf = lambda: ([pltpu.VMEM((CHUNK,), jnp.int32),
                    pltpu.VMEM((CHUNK,), jnp.int32),
                    pltpu.VMEM((CHUNK,), jnp.int32),
                    pltpu.VMEM((CHUNK, F), jnp.float32),
                    pltpu.SemaphoreType.DMA]
                   + ([pltpu.VMEM((CHUNK,), jnp.float32),
                       pltpu.SemaphoreType.DMA]
                      if scalar_mode == "dn" else []))
    scratch = buf() + buf() + [
        pltpu.VMEM_SHARED((NP, F), jnp.float32),
    ]
    if scalar_mode:
        scratch.append(pltpu.VMEM_SHARED((NP,), jnp.float32))
    if scalar_mode == "cnt":
        scratch.append(pltpu.VMEM((CHUNK,), jnp.float32))

    @functools.partial(pl.kernel, mesh=_mesh(), out_type=out_type,
                       scratch_types=scratch)
    def k(tab_hbm, he_hbm, zero_hbm, zero1_hbm, ones_hbm, nw_hbm, dep_hbm,
          *rest):
        if scalar_mode:
            acc_out, sc_out = rest[0], rest[1]
            rest = rest[2:]
        else:
            acc_out = rest[0]
            rest = rest[1:]
        bufs = (rest[0:nb], rest[nb:2 * nb])
        rest = rest[2 * nb:]
        acc_sh = rest[0]
        sacc_sh = rest[1] if scalar_mode else None
        ones_v = rest[2] if scalar_mode == "cnt" else None
        cid = lax.axis_index("c")
        sid = lax.axis_index("s")
        wid = sid * NC + cid
        goff = cid * N
        base0 = sid if stacked else wid

        tb = sid * rpt
        pltpu.sync_copy(zero_hbm.at[pl.ds(tb, rpt)], acc_sh.at[pl.ds(tb, rpt)])
        if scalar_mode == "cnt":
            pltpu.sync_copy(ones_hbm, ones_v)
        if scalar_mode:
            @pl.when((cid == 0) & (sid == 0))
            def _():
                pltpu.sync_copy(zero1_hbm, sacc_sh)
        plsc.subcore_barrier()

        def parts(b):
            t = bufs[b]
            w, sem2 = (t[5], t[6]) if nb == 7 else (None, None)
            return t[0], t[1], t[2], t[3], t[4], w, sem2

        def gref(b):
            gio, gim, si, rows, sem, w, sem2 = parts(b)
            return gim if stacked else gio

        def issue(i, b):
            gio, gim, si, rows, sem, w, sem2 = parts(b)
            ch = base0 + i * stride

            @pl.when(ch < NCHUNKS)
            def _():
                base = ch * CHUNK
                pltpu.sync_copy(he_hbm.at[gdim, pl.ds(base, CHUNK)], gio)
                pltpu.sync_copy(he_hbm.at[sdim, pl.ds(base, CHUNK)], si)
                if scalar_mode == "dn":
                    @pl.when(cid == 0)
                    def _():
                        pltpu.async_copy(nw_hbm.at[gio], w, sem2)
                if stacked:
                    for j in range(nsub):
                        sl = pl.ds(j * 16, 16)
                        gim[sl] = gio[sl] + goff
                pltpu.async_copy(tab_hbm.at[gref(b)], rows, sem)

        def drain(i, b):
            gio, gim, si, rows, sem, w, sem2 = parts(b)
            ch = base0 + i * stride

            @pl.when(ch < NCHUNKS)
            def _():
                pltpu.make_async_copy(tab_hbm.at[gref(b)], rows, sem).wait()
                pltpu.sync_copy(rows, acc_sh.at[si], add=True)
                if scalar_mode == "cnt":
                    @pl.when(cid == 0)
                    def _():
                        pltpu.sync_copy(ones_v, sacc_sh.at[si], add=True)
                elif scalar_mode == "dn":
                    @pl.when(cid == 0)
                    def _():
                        pltpu.make_async_copy(nw_hbm.at[gio], w, sem2).wait()
                        pltpu.sync_copy(w, sacc_sh.at[si], add=True)

        issue(0, 0)

        def body(i2, carry):
            i = 2 * i2
            issue(i + 1, 1)
            drain(i, 0)
            issue(i + 2, 0)
            drain(i + 1, 1)
            return carry

        lax.fori_loop(0, np2, body, 0)
        plsc.subcore_barrier()
        pltpu.sync_copy(acc_sh.at[pl.ds(tb, rpt)],
                        acc_out.at[cid, pl.ds(tb, rpt)])
        if scalar_mode:
            @pl.when(cid == 0)
            def _():
                pltpu.sync_copy(sacc_sh.at[pl.ds(tb, rpt)],
                                sc_out.at[pl.ds(tb, rpt)])

    zeros = jnp.zeros((NP, F), jnp.float32)
    zeros1 = jnp.zeros((NP,), jnp.float32)
    ones = jnp.ones((CHUNK,), jnp.float32)
    if nw is None:
        nw = jnp.zeros((N,), jnp.float32)
    if sched_dep is None:
        sched_dep = ones
    res = k(table, hyper_edge, zeros, zeros1, ones, nw, sched_dep)
    return res if scalar_mode else res[0]


NBLK = 2000
NNB = N // NBLK

_full = lambda shape: pl.BlockSpec(shape, lambda i: (0,) * len(shape))
_nrow = lambda w: pl.BlockSpec((NBLK, w), lambda i: (i, 0))


def _node_body(gemb_tab_ref, whl1a_ref, whl1b_ref, bhl1_ref, whl2_ref,
               bhl2_ref, x_ref, batch_ref, nw_ref):
    b = batch_ref[0, 0]
    onehot = (b[:, None] == lax.broadcasted_iota(jnp.int32, (1, G), 1)
              ).astype(jnp.float32)
    proto = jnp.dot(onehot, gemb_tab_ref[...],
                    preferred_element_type=jnp.float32)
    h = jnp.dot(x_ref[...], whl1a_ref[...], preferred_element_type=jnp.float32)
    h += jnp.dot(proto, whl1b_ref[...], preferred_element_type=jnp.float32)
    h = jnp.maximum(h + bhl1_ref[...], 0.0)
    nw = jnp.dot(h, whl2_ref[...], preferred_element_type=jnp.float32) \
        + bhl2_ref[...]
    nw_ref[...] = jax.nn.sigmoid(nw)


def _node_stage(x, graph_emb, batch, Whl1, bhl1, Whl2, bhl2):
    return pl.pallas_call(
        _node_body,
        grid=(NNB,),
        in_specs=[
            _full((G, F)), _full((F, F)), _full((F, F)), _full((1, F)),
            _full((F, 1)), _full((1, 1)),
            _nrow(F),
            pl.BlockSpec((1, 1, NBLK), lambda i: (i, 0, 0)),
        ],
        out_specs=_nrow(1),
        out_shape=jax.ShapeDtypeStruct((N, 1), jnp.float32),
    )(graph_emb, Whl1[:F], Whl1[F:], bhl1.reshape(1, -1), Whl2,
      bhl2.reshape(1, -1), x, batch.reshape(NNB, 1, NBLK))


_stk = pl.BlockSpec((2, NBLK, F), lambda i: (0, i, 0))
_scal = pl.BlockSpec((NBLK, 1), lambda i: (i, 0))


def _prep1_body(wa_ref, wb_ref, b_ref, gxc_ref, gxr_ref, out_ref):
    Xl = jnp.dot(gxc_ref[...], wa_ref[...], preferred_element_type=jnp.float32)
    Xl += jnp.dot(gxr_ref[...], wb_ref[...], preferred_element_type=jnp.float32)
    Xl += b_ref[...]
    out_ref[...] = jnp.stack([Xl[:, :F], Xl[:, F:]], axis=0)


def _prep1(gxc, gxr, Whc1, bhc1):
    return pl.pallas_call(
        _prep1_body,
        grid=(NNB,),
        in_specs=[_full((F, 2 * F)), _full((F, 2 * F)), _full((1, 2 * F)),
                  _nrow(F), _nrow(F)],
        out_specs=_stk,
        out_shape=jax.ShapeDtypeStruct((2, N, F), jnp.float32),
    )(Whc1[:F], Whc1[F:], bhc1.reshape(1, -1), gxc, gxr)


def _prep2_body(s1_ref, cntp_ref, nw_ref, out_ref, scale_ref):
    cnt = cntp_ref[...][:, 0]
    nw = nw_ref[...][:, 0]
    s = jnp.where(cnt > 0, nw / cnt, 0.0)
    out_ref[...] = s[None, :, None] * s1_ref[...]
    scale_ref[...] = s[:, None]


def _prep2(S1, cntp, nw):
    return pl.pallas_call(
        _prep2_body,
        grid=(NNB,),
        in_specs=[_stk, _scal, _nrow(1)],
        out_specs=[_stk, _nrow(1)],
        out_shape=[jax.ShapeDtypeStruct((2, N, F), jnp.float32),
                   jax.ShapeDtypeStruct((N, 1), jnp.float32)],
    )(S1, cntp, nw)


def _prep3_body(whc2a_ref, whc2b_ref, b_ref, t1_ref, dnp_ref,
                out_ref, dinv_ref):
    dn = dnp_ref[...][:, 0]
    dinv = jnp.where(dn > 0, 1.0 / dn, 0.0)
    era = jax.nn.sigmoid(dinv[:, None] * t1_ref[0])
    erb = jax.nn.sigmoid(dinv[:, None] * t1_ref[1])
    Xl2 = jnp.dot(era, whc2a_ref[...], preferred_element_type=jnp.float32)
    Xl2 += jnp.dot(erb, whc2b_ref[...], preferred_element_type=jnp.float32)
    out_ref[...] = Xl2 + b_ref[...]
    dinv_ref[...] = dinv[:, None]


def _prep3(T1, dnp, Whc2, bhc2):
    return pl.pallas_call(
        _prep3_body,
        grid=(NNB,),
        in_specs=[_full((F, F)), _full((F, F)), _full((1, F)), _stk, _scal],
        out_specs=[_nrow(F), _nrow(1)],
        out_shape=[jax.ShapeDtypeStruct((N, F), jnp.float32),
                   jax.ShapeDtypeStruct((N, 1), jnp.float32)],
    )(Whc2[:F], Whc2[F:], bhc2.reshape(1, -1), T1, dnp)


def _prep4_body(s2_ref, scale_ref, out_ref):
    out_ref[...] = scale_ref[...] * (s2_ref[0] + s2_ref[1])


def _prep4(S2, scale):
    return pl.pallas_call(
        _prep4_body,
        grid=(NNB,),
        in_specs=[_stk, _nrow(1)],
        out_specs=_nrow(F),
        out_shape=jax.ShapeDtypeStruct((N, F), jnp.float32),
    )(S2, scale)


def _prep5_body(t2_ref, dinv_ref, out_ref):
    out_ref[...] = jax.nn.sigmoid(dinv_ref[...] * (t2_ref[0] + t2_ref[1]))


def _prep5(T2, dinv):
    return pl.pallas_call(
        _prep5_body,
        grid=(NNB,),
        in_specs=[_stk, _nrow(1)],
        out_specs=_nrow(F),
        out_shape=jax.ShapeDtypeStruct((N, F), jnp.float32),
    )(T2, dinv)


EBLK = 2000
NEB = E // EBLK
NSH = N // EBLK


def _edge_heavy_body(gemb_tab_ref, wl1a_ref, wl1b_ref, wl1c_ref, bl1_ref,
                     wl2_ref, bl2_ref, gxc_ref, gxr_ref, bcol_ref, out_ref):
    bcol = bcol_ref[0, 0]
    onehot = (bcol[:, None] == lax.broadcasted_iota(jnp.int32, (1, G), 1)
              ).astype(jnp.float32)
    gemb = jnp.dot(onehot, gemb_tab_ref[...],
                   preferred_element_type=jnp.float32)
    h1 = jnp.dot(gxc_ref[...], wl1a_ref[...], preferred_element_type=jnp.float32)
    h1 += jnp.dot(gxr_ref[...], wl1b_ref[...], preferred_element_type=jnp.float32)
    h1 += jnp.dot(gemb, wl1c_ref[...], preferred_element_type=jnp.float32)
    h1 = jnp.maximum(h1 + bl1_ref[...], 0.0)
    out_ref[...] = jnp.maximum(
        jnp.dot(h1, wl2_ref[...], preferred_element_type=jnp.float32)
        + bl2_ref[...], 0.0)


def _edge_heavy(gxc, gxr, bcol, graph_emb, Wl1, bl1, Wl2, bl2):
    return pl.pallas_call(
        _edge_heavy_body,
        grid=(NEB,),
        in_specs=[
            _full((G, F)),
            _full((F, 4 * F)), _full((F, 4 * F)), _full((F, 4 * F)),
            _full((1, 4 * F)),
            _full((4 * F, F)), _full((1, F)),
            pl.BlockSpec((EBLK, F), lambda i: (i, 0)),
            pl.BlockSpec((EBLK, F), lambda i: (i, 0)),
            pl.BlockSpec((1, 1, EBLK), lambda i: (i, 0, 0)),
        ],
        out_specs=pl.BlockSpec((EBLK, F), lambda i: (i, 0)),
        out_shape=jax.ShapeDtypeStruct((E, F), jnp.float32),
    )(graph_emb, Wl1[:F], Wl1[F:2 * F], Wl1[2 * F:], bl1.reshape(1, -1),
      Wl2, bl2.reshape(1, -1), gxc, gxr, bcol.reshape(NEB, 1, EBLK))


def _edge_light_body(gemb_tab_ref, wc1a_ref, wc1b_ref, bc1_ref,
                     wc2_ref, bc2_ref, attn_ref,
                     xij2_ref, bcol_ref, t2_ref, dinv_ref, out_ref):
    pid = pl.program_id(0)
    bcol = bcol_ref[0, 0]
    onehot = (bcol[:, None] == lax.broadcasted_iota(jnp.int32, (1, G), 1)
              ).astype(jnp.float32)
    gemb = jnp.dot(onehot, gemb_tab_ref[...],
                   preferred_element_type=jnp.float32)
    sh_n = jax.nn.sigmoid(dinv_ref[...] * (t2_ref[0] + t2_ref[1]))
    sh = jnp.where(pid < NSH, sh_n, 0.5)
    s = attn_ref[0, 0] * xij2_ref[...] + attn_ref[0, 1] * sh
    z = jnp.dot(s, wc1a_ref[...], preferred_element_type=jnp.float32)
    z += jnp.dot(gemb, wc1b_ref[...], preferred_element_type=jnp.float32)
    z = jnp.maximum(z + bc1_ref[...], 0.0)
    o = jnp.dot(z, wc2_ref[...], preferred_element_type=jnp.float32) \
        + bc2_ref[...]
    out_ref[...] = jax.nn.sigmoid(o)


def _edge_light(xij2, bcol, T2, dinv, graph_emb, Wc1, bc1, Wc2, bc2, attn):
    clamp = lambda i: jnp.minimum(i, NSH - 1)
    return pl.pallas_call(
        _edge_light_body,
        grid=(NEB,),
        in_specs=[
            _full((G, F)),
            _full((F, F)), _full((F, F)), _full((1, F)),
            _full((F, 1)), _full((1, 1)),
            _full((1, 2)),
            pl.BlockSpec((EBLK, F), lambda i: (i, 0)),
            pl.BlockSpec((1, 1, EBLK), lambda i: (i, 0, 0)),
            pl.BlockSpec((2, EBLK, F), lambda i: (0, clamp(i), 0)),
            pl.BlockSpec((EBLK, 1), lambda i: (clamp(i), 0)),
        ],
        out_specs=pl.BlockSpec((EBLK, 1), lambda i: (i, 0)),
        out_shape=jax.ShapeDtypeStruct((E, 1), jnp.float32),
    )(graph_emb, Wc1[:F], Wc1[F:], bc1.reshape(1, -1),
      Wc2, bc2.reshape(1, -1), attn.reshape(1, 2),
      xij2, bcol.reshape(NEB, 1, EBLK), T2, dinv)


def kernel(x, graph_emb, edge_index, edge_type, batch, hyper_edge, attn,
           Whl1, bhl1, Whl2, bhl2, Whc1, bhc1, Whc2, bhc2,
           Wl1, bl1, Wl2, bl2, Wc1, bc1, Wc2, bc2):
    nw = _node_stage(x, graph_emb, batch, Whl1, bhl1, Whl2, bhl2)

    gxc, gxr, bcol = _sc_edge_gather(x, edge_index, batch)

    xij2 = _edge_heavy(gxc, gxr, bcol, graph_emb, Wl1, bl1, Wl2, bl2)

    Xl1e = _prep1(gxc, gxr, Whc1, bhc1)
    S1, cntp = _sc_hconv_pass(Xl1e.reshape(2 * N, F), hyper_edge, 0, 1,
                              stacked=True, scalar_mode="cnt")
    a1e, scale = _prep2(S1, cntp.reshape(NP, 1), nw)
    T1, dnp = _sc_hconv_pass(a1e.reshape(2 * N, F), hyper_edge, 1, 0,
                             stacked=True, scalar_mode="dn", nw=nw[:, 0],
                             sched_dep=xij2)
    Xl2, dinv = _prep3(T1, dnp.reshape(NP, 1), Whc2, bhc2)
    S2 = _sc_hconv_pass(Xl2, hyper_edge, 0, 1, stacked=False)
    a2 = _prep4(S2, scale)
    T2 = _sc_hconv_pass(a2, hyper_edge, 1, 0, stacked=False)

    sij = _edge_light(xij2, bcol, T2, dinv, graph_emb,
                      Wc1, bc1, Wc2, bc2, attn)
    return (edge_index, edge_type, sij)


# speedup vs baseline: 1.1775x; 1.1023x over previous
import functools

import jax
import jax.numpy as jnp
from jax import lax
from jax.experimental import pallas as pl
from jax.experimental.pallas import tpu as pltpu
from jax.experimental.pallas import tpu_sc as plsc

N = 10000
E = 160000
NNZ = 320000
G = 64
F = 128

NC = 2
NS = 16
NTILES = NC * NS
CHUNK = 128
CEXT = 144
CH2 = 64

_mesh = lambda: plsc.VectorSubcoreMesh(core_axis_name="c", subcore_axis_name="s")


def _sc_edge_gather(x, edge_index, batch):
    nchunks = E // CHUNK
    npt = -(-nchunks // NTILES)
    np3 = -(-npt // 3)

    buf = lambda: [pltpu.VMEM((CHUNK,), jnp.int32),
                   pltpu.VMEM((CHUNK,), jnp.int32),
                   pltpu.VMEM((CHUNK, F), jnp.float32),
                   pltpu.VMEM((CHUNK, F), jnp.float32),
                   pltpu.VMEM((CHUNK,), jnp.int32),
                   pltpu.SemaphoreType.DMA,
                   pltpu.SemaphoreType.DMA,
                   pltpu.SemaphoreType.DMA]

    @functools.partial(
        pl.kernel,
        mesh=_mesh(),
        out_type=[
            jax.ShapeDtypeStruct((E, F), jnp.float32),
            jax.ShapeDtypeStruct((E, F), jnp.float32),
            jax.ShapeDtypeStruct((E,), jnp.int32),
        ],
        scratch_types=buf() + buf() + buf(),
    )
    def k(x_hbm, eidx_hbm, batch_hbm, gxc_hbm, gxr_hbm, bcol_hbm,
          *rest):
        wid = lax.axis_index("s") * NC + lax.axis_index("c")
        bufs = (rest[0:8], rest[8:16], rest[16:24])

        def issue(i, b):
            ci, ri, rc, rr, bv, s1, s2, s3 = bufs[b]
            ch = wid + i * NTILES

            @pl.when(ch < nchunks)
            def _():
                base = ch * CHUNK
                pltpu.sync_copy(eidx_hbm.at[0, pl.ds(base, CHUNK)], ci)
                pltpu.sync_copy(eidx_hbm.at[1, pl.ds(base, CHUNK)], ri)
                pltpu.async_copy(x_hbm.at[ci], rc, s1)
                pltpu.async_copy(x_hbm.at[ri], rr, s2)
                pltpu.async_copy(batch_hbm.at[ci], bv, s3)

        def drain(i, b):
            ci, ri, rc, rr, bv, s1, s2, s3 = bufs[b]
            ch = wid + i * NTILES

            @pl.when(ch < nchunks)
            def _():
                base = ch * CHUNK
                pltpu.make_async_copy(x_hbm.at[ci], rc, s1).wait()
                pltpu.make_async_copy(x_hbm.at[ri], rr, s2).wait()
                pltpu.make_async_copy(batch_hbm.at[ci], bv, s3).wait()
                pltpu.sync_copy(rc, gxc_hbm.at[pl.ds(base, CHUNK)])
                pltpu.sync_copy(rr, gxr_hbm.at[pl.ds(base, CHUNK)])
                pltpu.sync_copy(bv, bcol_hbm.at[pl.ds(base, CHUNK)])

        issue(0, 0)
        issue(1, 1)

        def body(i3, carry):
            i = 3 * i3
            issue(i + 2, 2)
            drain(i, 0)
            issue(i + 3, 0)
            drain(i + 1, 1)
            issue(i + 4, 1)
            drain(i + 2, 2)
            return carry

        lax.fori_loop(0, np3, body, 0)

    return k(x, edge_index, batch)


NP = 10240
NCHUNKS = NNZ // CHUNK


def _sc_hconv_pass(table, hyper_edge, gdim, sdim, stacked,
                   scalar_mode=None, nw=None, sched_dep=None):
    stride = NS if stacked else NTILES
    npt = -(-NCHUNKS // stride)
    np2 = -(-npt // 2)
    rpt = NP // NS
    nsub = CHUNK // 16

    out_type = [jax.ShapeDtypeStruct((NC, NP, F), jnp.float32)]
    if scalar_mode:
        out_type.append(jax.ShapeDtypeStruct((NP,), jnp.float32))

    nb = 7 if scalar_mode == "dn" else 5
    buf = lambda: ([pltpu.VMEM((CHUNK,), jnp.int32),
                    pltpu.VMEM((CHUNK,), jnp.int32),
                    pltpu.VMEM((CHUNK,), jnp.int32),
                    pltpu.VMEM((CHUNK, F), jnp.float32),
                    pltpu.SemaphoreType.DMA]
                   + ([pltpu.VMEM((CHUNK,), jnp.float32),
                       pltpu.SemaphoreType.DMA]
                      if scalar_mode == "dn" else []))
    scratch = buf() + buf() + [
        pltpu.VMEM_SHARED((NP, F), jnp.float32),
    ]
    if scalar_mode:
        scratch.append(pltpu.VMEM_SHARED((NP,), jnp.float32))
    if scalar_mode == "cnt":
        scratch.append(pltpu.VMEM((CHUNK,), jnp.float32))

    @functools.partial(pl.kernel, mesh=_mesh(), out_type=out_type,
                       scratch_types=scratch)
    def k(tab_hbm, he_hbm, zero_hbm, zero1_hbm, ones_hbm, nw_hbm, dep_hbm,
          *rest):
        if scalar_mode:
            acc_out, sc_out = rest[0], rest[1]
            rest = rest[2:]
        else:
            acc_out = rest[0]
            rest = rest[1:]
        bufs = (rest[0:nb], rest[nb:2 * nb])
        rest = rest[2 * nb:]
        acc_sh = rest[0]
        sacc_sh = rest[1] if scalar_mode else None
        ones_v = rest[2] if scalar_mode == "cnt" else None
        cid = lax.axis_index("c")
        sid = lax.axis_index("s")
        wid = sid * NC + cid
        goff = cid * N
        base0 = sid if stacked else wid

        tb = sid * rpt
        pltpu.sync_copy(zero_hbm.at[pl.ds(tb, rpt)], acc_sh.at[pl.ds(tb, rpt)])
        if scalar_mode == "cnt":
            pltpu.sync_copy(ones_hbm, ones_v)
        if scalar_mode:
            @pl.when((cid == 0) & (sid == 0))
            def _():
                pltpu.sync_copy(zero1_hbm, sacc_sh)
        plsc.subcore_barrier()

        def parts(b):
            t = bufs[b]
            w, sem2 = (t[5], t[6]) if nb == 7 else (None, None)
            return t[0], t[1], t[2], t[3], t[4], w, sem2

        def gref(b):
            gio, gim, si, rows, sem, w, sem2 = parts(b)
            return gim if stacked else gio

        def issue(i, b):
            gio, gim, si, rows, sem, w, sem2 = parts(b)
            ch = base0 + i * stride

            @pl.when(ch < NCHUNKS)
            def _():
                base = ch * CHUNK
                pltpu.sync_copy(he_hbm.at[gdim, pl.ds(base, CHUNK)], gio)
                pltpu.sync_copy(he_hbm.at[sdim, pl.ds(base, CHUNK)], si)
                if scalar_mode == "dn":
                    @pl.when(cid == 0)
                    def _():
                        pltpu.async_copy(nw_hbm.at[gio], w, sem2)
                if stacked:
                    for j in range(nsub):
                        sl = pl.ds(j * 16, 16)
                        gim[sl] = gio[sl] + goff
                pltpu.async_copy(tab_hbm.at[gref(b)], rows, sem)

        def drain(i, b):
            gio, gim, si, rows, sem, w, sem2 = parts(b)
            ch = base0 + i * stride

            @pl.when(ch < NCHUNKS)
            def _():
                pltpu.make_async_copy(tab_hbm.at[gref(b)], rows, sem).wait()
                pltpu.sync_copy(rows, acc_sh.at[si], add=True)
                if scalar_mode == "cnt":
                    @pl.when(cid == 0)
                    def _():
                        pltpu.sync_copy(ones_v, sacc_sh.at[si], add=True)
                elif scalar_mode == "dn":
                    @pl.when(cid == 0)
                    def _():
                        pltpu.make_async_copy(nw_hbm.at[gio], w, sem2).wait()
                        pltpu.sync_copy(w, sacc_sh.at[si], add=True)

        issue(0, 0)

        def body(i2, carry):
            i = 2 * i2
            issue(i + 1, 1)
            drain(i, 0)
            issue(i + 2, 0)
            drain(i + 1, 1)
            return carry

        lax.fori_loop(0, np2, body, 0)
        plsc.subcore_barrier()
        pltpu.sync_copy(acc_sh.at[pl.ds(tb, rpt)],
                        acc_out.at[cid, pl.ds(tb, rpt)])
        if scalar_mode:
            @pl.when(cid == 0)
            def _():
                pltpu.sync_copy(sacc_sh.at[pl.ds(tb, rpt)],
                                sc_out.at[pl.ds(tb, rpt)])

    zeros = jnp.zeros((NP, F), jnp.float32)
    zeros1 = jnp.zeros((NP,), jnp.float32)
    ones = jnp.ones((CHUNK,), jnp.float32)
    if nw is None:
        nw = jnp.zeros((N,), jnp.float32)
    if sched_dep is None:
        sched_dep = ones
    res = k(table, hyper_edge, zeros, zeros1, ones, nw, sched_dep)
    return res if scalar_mode else res[0]


NBLK = 2000
NNB = N // NBLK

_full = lambda shape: pl.BlockSpec(shape, lambda i: (0,) * len(shape))
_nrow = lambda w: pl.BlockSpec((NBLK, w), lambda i: (i, 0))


def _node_body(gemb_tab_ref, whl1a_ref, whl1b_ref, bhl1_ref, whl2_ref,
               bhl2_ref, x_ref, batch_ref, nw_ref):
    b = batch_ref[0, 0]
    onehot = (b[:, None] == lax.broadcasted_iota(jnp.int32, (1, G), 1)
              ).astype(jnp.float32)
    proto = jnp.dot(onehot, gemb_tab_ref[...],
                    preferred_element_type=jnp.float32)
    h = jnp.dot(x_ref[...], whl1a_ref[...], preferred_element_type=jnp.float32)
    h += jnp.dot(proto, whl1b_ref[...], preferred_element_type=jnp.float32)
    h = jnp.maximum(h + bhl1_ref[...], 0.0)
    nw = jnp.dot(h, whl2_ref[...], preferred_element_type=jnp.float32) \
        + bhl2_ref[...]
    nw_ref[...] = jax.nn.sigmoid(nw)


def _node_stage(x, graph_emb, batch, Whl1, bhl1, Whl2, bhl2):
    return pl.pallas_call(
        _node_body,
        grid=(NNB,),
        in_specs=[
            _full((G, F)), _full((F, F)), _full((F, F)), _full((1, F)),
            _full((F, 1)), _full((1, 1)),
            _nrow(F),
            pl.BlockSpec((1, 1, NBLK), lambda i: (i, 0, 0)),
        ],
        out_specs=_nrow(1),
        out_shape=jax.ShapeDtypeStruct((N, 1), jnp.float32),
    )(graph_emb, Whl1[:F], Whl1[F:], bhl1.reshape(1, -1), Whl2,
      bhl2.reshape(1, -1), x, batch.reshape(NNB, 1, NBLK))


_stk = pl.BlockSpec((2, NBLK, F), lambda i: (0, i, 0))
_scal = pl.BlockSpec((NBLK, 1), lambda i: (i, 0))


def _prep1_body(wa_ref, wb_ref, b_ref, gxc_ref, gxr_ref, out_ref):
    Xl = jnp.dot(gxc_ref[...], wa_ref[...], preferred_element_type=jnp.float32)
    Xl += jnp.dot(gxr_ref[...], wb_ref[...], preferred_element_type=jnp.float32)
    Xl += b_ref[...]
    out_ref[...] = jnp.stack([Xl[:, :F], Xl[:, F:]], axis=0)


def _prep1(gxc, gxr, Whc1, bhc1):
    return pl.pallas_call(
        _prep1_body,
        grid=(NNB,),
        in_specs=[_full((F, 2 * F)), _full((F, 2 * F)), _full((1, 2 * F)),
                  _nrow(F), _nrow(F)],
        out_specs=_stk,
        out_shape=jax.ShapeDtypeStruct((2, N, F), jnp.float32),
    )(Whc1[:F], Whc1[F:], bhc1.reshape(1, -1), gxc, gxr)


def _prep2_body(s1_ref, cntp_ref, nw_ref, out_ref, scale_ref):
    cnt = cntp_ref[...][:, 0]
    nw = nw_ref[...][:, 0]
    s = jnp.where(cnt > 0, nw / cnt, 0.0)
    out_ref[...] = s[None, :, None] * s1_ref[...]
    scale_ref[...] = s[:, None]


def _prep2(S1, cntp, nw):
    return pl.pallas_call(
        _prep2_body,
        grid=(NNB,),
        in_specs=[_stk, _scal, _nrow(1)],
        out_specs=[_stk, _nrow(1)],
        out_shape=[jax.ShapeDtypeStruct((2, N, F), jnp.float32),
                   jax.ShapeDtypeStruct((N, 1), jnp.float32)],
    )(S1, cntp, nw)


def _prep3_body(whc2a_ref, whc2b_ref, b_ref, t1_ref, dnp_ref,
                out_ref, dinv_ref):
    dn = dnp_ref[...][:, 0]
    dinv = jnp.where(dn > 0, 1.0 / dn, 0.0)
    era = jax.nn.sigmoid(dinv[:, None] * t1_ref[0])
    erb = jax.nn.sigmoid(dinv[:, None] * t1_ref[1])
    Xl2 = jnp.dot(era, whc2a_ref[...], preferred_element_type=jnp.float32)
    Xl2 += jnp.dot(erb, whc2b_ref[...], preferred_element_type=jnp.float32)
    out_ref[...] = Xl2 + b_ref[...]
    dinv_ref[...] = dinv[:, None]


def _prep3(T1, dnp, Whc2, bhc2):
    return pl.pallas_call(
        _prep3_body,
        grid=(NNB,),
        in_specs=[_full((F, F)), _full((F, F)), _full((1, F)), _stk, _scal],
        out_specs=[_nrow(F), _nrow(1)],
        out_shape=[jax.ShapeDtypeStruct((N, F), jnp.float32),
                   jax.ShapeDtypeStruct((N, 1), jnp.float32)],
    )(Whc2[:F], Whc2[F:], bhc2.reshape(1, -1), T1, dnp)


def _prep4_body(s2_ref, scale_ref, out_ref):
    out_ref[...] = scale_ref[...] * (s2_ref[0] + s2_ref[1])


def _prep4(S2, scale):
    return pl.pallas_call(
        _prep4_body,
        grid=(NNB,),
        in_specs=[_stk, _nrow(1)],
        out_specs=_nrow(F),
        out_shape=jax.ShapeDtypeStruct((N, F), jnp.float32),
    )(S2, scale)


def _prep5_body(t2_ref, dinv_ref, out_ref):
    out_ref[...] = jax.nn.sigmoid(dinv_ref[...] * (t2_ref[0] + t2_ref[1]))


def _prep5(T2, dinv):
    return pl.pallas_call(
        _prep5_body,
        grid=(NNB,),
        in_specs=[_stk, _nrow(1)],
        out_specs=_nrow(F),
        out_shape=jax.ShapeDtypeStruct((N, F), jnp.float32),
    )(T2, dinv)


EBLK = 2000
NEB = E // EBLK
NSH = N // EBLK


def _edge_heavy_body(gemb_tab_ref, wl1a_ref, wl1b_ref, wl1c_ref, bl1_ref,
                     wl2_ref, bl2_ref, gxc_ref, gxr_ref, bcol_ref, out_ref):
    bcol = bcol_ref[0, 0]
    onehot = (bcol[:, None] == lax.broadcasted_iota(jnp.int32, (1, G), 1)
              ).astype(jnp.float32)
    gemb = jnp.dot(onehot, gemb_tab_ref[...],
                   preferred_element_type=jnp.float32)
    h1 = jnp.dot(gxc_ref[...], wl1a_ref[...], preferred_element_type=jnp.float32)
    h1 += jnp.dot(gxr_ref[...], wl1b_ref[...], preferred_element_type=jnp.float32)
    h1 += jnp.dot(gemb, wl1c_ref[...], preferred_element_type=jnp.float32)
    h1 = jnp.maximum(h1 + bl1_ref[...], 0.0)
    out_ref[...] = jnp.maximum(
        jnp.dot(h1, wl2_ref[...], preferred_element_type=jnp.float32)
        + bl2_ref[...], 0.0)


def _edge_heavy(gxc, gxr, bcol, graph_emb, Wl1, bl1, Wl2, bl2):
    return pl.pallas_call(
        _edge_heavy_body,
        grid=(NEB,),
        in_specs=[
            _full((G, F)),
            _full((F, 4 * F)), _full((F, 4 * F)), _full((F, 4 * F)),
            _full((1, 4 * F)),
            _full((4 * F, F)), _full((1, F)),
            pl.BlockSpec((EBLK, F), lambda i: (i, 0)),
            pl.BlockSpec((EBLK, F), lambda i: (i, 0)),
            pl.BlockSpec((1, 1, EBLK), lambda i: (i, 0, 0)),
        ],
        out_specs=pl.BlockSpec((EBLK, F), lambda i: (i, 0)),
        out_shape=jax.ShapeDtypeStruct((E, F), jnp.float32),
    )(graph_emb, Wl1[:F], Wl1[F:2 * F], Wl1[2 * F:], bl1.reshape(1, -1),
      Wl2, bl2.reshape(1, -1), gxc, gxr, bcol.reshape(NEB, 1, EBLK))


def _light_core(gemb_tab_ref, wc1a_ref, wc1b_ref, bc1_ref, wc2_ref, bc2_ref,
                attn_ref, xij2_ref, bcol_ref, sh, out_ref):
    bcol = bcol_ref[0, 0]
    onehot = (bcol[:, None] == lax.broadcasted_iota(jnp.int32, (1, G), 1)
              ).astype(jnp.float32)
    gemb = jnp.dot(onehot, gemb_tab_ref[...],
                   preferred_element_type=jnp.float32)
    s = attn_ref[0, 0] * xij2_ref[...] + attn_ref[0, 1] * sh
    z = jnp.dot(s, wc1a_ref[...], preferred_element_type=jnp.float32)
    z += jnp.dot(gemb, wc1b_ref[...], preferred_element_type=jnp.float32)
    z = jnp.maximum(z + bc1_ref[...], 0.0)
    o = jnp.dot(z, wc2_ref[...], preferred_element_type=jnp.float32) \
        + bc2_ref[...]
    out_ref[...] = jax.nn.sigmoid(o)


def _light_main_body(gemb_tab_ref, wc1a_ref, wc1b_ref, bc1_ref,
                     wc2_ref, bc2_ref, attn_ref, xij2_ref, bcol_ref, out_ref):
    _light_core(gemb_tab_ref, wc1a_ref, wc1b_ref, bc1_ref, wc2_ref, bc2_ref,
                attn_ref, xij2_ref, bcol_ref, 0.5, out_ref)


def _light_pref_body(gemb_tab_ref, wc1a_ref, wc1b_ref, bc1_ref,
                     wc2_ref, bc2_ref, attn_ref, xij2_ref, bcol_ref,
                     t2_ref, dinv_ref, out_ref):
    sh = jax.nn.sigmoid(dinv_ref[...] * (t2_ref[0] + t2_ref[1]))
    _light_core(gemb_tab_ref, wc1a_ref, wc1b_ref, bc1_ref, wc2_ref, bc2_ref,
                attn_ref, xij2_ref, bcol_ref, sh, out_ref)


_LIGHT_W = lambda: [_full((G, F)), _full((F, F)), _full((F, F)),
                    _full((1, F)), _full((F, 1)), _full((1, 1)),
                    _full((1, 2))]


def _light_wargs(graph_emb, Wc1, bc1, Wc2, bc2, attn):
    return (graph_emb, Wc1[:F], Wc1[F:], bc1.reshape(1, -1),
            Wc2, bc2.reshape(1, -1), attn.reshape(1, 2))


def _edge_light_main(xij2, bcol, w_args):
    return pl.pallas_call(
        _light_main_body,
        grid=(NEB - NSH,),
        in_specs=_LIGHT_W() + [
            pl.BlockSpec((EBLK, F), lambda i: (i + NSH, 0)),
            pl.BlockSpec((1, 1, EBLK), lambda i: (i + NSH, 0, 0)),
        ],
        out_specs=pl.BlockSpec((EBLK, 1), lambda i: (i, 0)),
        out_shape=jax.ShapeDtypeStruct((E - N, 1), jnp.float32),
    )(*w_args, xij2, bcol.reshape(NEB, 1, EBLK))


def _edge_light_prefix(xij2, bcol, T2, dinv, w_args):
    return pl.pallas_call(
        _light_pref_body,
        grid=(NSH,),
        in_specs=_LIGHT_W() + [
            pl.BlockSpec((EBLK, F), lambda i: (i, 0)),
            pl.BlockSpec((1, 1, EBLK), lambda i: (i, 0, 0)),
            pl.BlockSpec((2, EBLK, F), lambda i: (0, i, 0)),
            pl.BlockSpec((EBLK, 1), lambda i: (i, 0)),
        ],
        out_specs=pl.BlockSpec((EBLK, 1), lambda i: (i, 0)),
        out_shape=jax.ShapeDtypeStruct((N, 1), jnp.float32),
    )(*w_args, xij2, bcol.reshape(NEB, 1, EBLK), T2, dinv)


def kernel(x, graph_emb, edge_index, edge_type, batch, hyper_edge, attn,
           Whl1, bhl1, Whl2, bhl2, Whc1, bhc1, Whc2, bhc2,
           Wl1, bl1, Wl2, bl2, Wc1, bc1, Wc2, bc2):
    nw = _node_stage(x, graph_emb, batch, Whl1, bhl1, Whl2, bhl2)

    gxc, gxr, bcol = _sc_edge_gather(x, edge_index, batch)

    xij2 = _edge_heavy(gxc, gxr, bcol, graph_emb, Wl1, bl1, Wl2, bl2)

    Xl1e = _prep1(gxc, gxr, Whc1, bhc1)
    S1, cntp = _sc_hconv_pass(Xl1e.reshape(2 * N, F), hyper_edge, 0, 1,
                              stacked=True, scalar_mode="cnt")
    a1e, scale = _prep2(S1, cntp.reshape(NP, 1), nw)
    T1, dnp = _sc_hconv_pass(a1e.reshape(2 * N, F), hyper_edge, 1, 0,
                             stacked=True, scalar_mode="dn", nw=nw[:, 0],
                             sched_dep=xij2)
    w_args = _light_wargs(graph_emb, Wc1, bc1, Wc2, bc2, attn)
    sij_main = _edge_light_main(xij2, bcol, w_args)

    Xl2, dinv = _prep3(T1, dnp.reshape(NP, 1), Whc2, bhc2)
    S2 = _sc_hconv_pass(Xl2, hyper_edge, 0, 1, stacked=False,
                        sched_dep=sij_main)
    a2 = _prep4(S2, scale)
    T2 = _sc_hconv_pass(a2, hyper_edge, 1, 0, stacked=False)

    sij_pref = _edge_light_prefix(xij2, bcol, T2, dinv, w_args)
    sij = jnp.concatenate([sij_pref, sij_main], axis=0)
    return (edge_index, edge_type, sij)


# speedup vs baseline: 1.3924x; 1.1825x over previous
import functools

import jax
import jax.numpy as jnp
from jax import lax
from jax.experimental import pallas as pl
from jax.experimental.pallas import tpu as pltpu
from jax.experimental.pallas import tpu_sc as plsc

N = 10000
E = 160000
NNZ = 320000
G = 64
F = 128

NC = 2
NS = 16
NTILES = NC * NS
CHUNK = 128
CEXT = 144
CH2 = 64

_mesh = lambda: plsc.VectorSubcoreMesh(core_axis_name="c", subcore_axis_name="s")


def _sc_edge_gather(x, edge_index, batch):
    nchunks = E // CHUNK
    npt = -(-nchunks // NTILES)
    np3 = -(-npt // 3)

    buf = lambda: [pltpu.VMEM((CHUNK,), jnp.int32),
                   pltpu.VMEM((CHUNK,), jnp.int32),
                   pltpu.VMEM((CHUNK, F), jnp.float32),
                   pltpu.VMEM((CHUNK, F), jnp.float32),
                   pltpu.VMEM((CHUNK,), jnp.int32),
                   pltpu.SemaphoreType.DMA,
                   pltpu.SemaphoreType.DMA,
                   pltpu.SemaphoreType.DMA]

    @functools.partial(
        pl.kernel,
        mesh=_mesh(),
        out_type=[
            jax.ShapeDtypeStruct((E, F), jnp.float32),
            jax.ShapeDtypeStruct((E, F), jnp.float32),
            jax.ShapeDtypeStruct((E,), jnp.int32),
        ],
        scratch_types=buf() + buf() + buf(),
    )
    def k(x_hbm, eidx_hbm, batch_hbm, gxc_hbm, gxr_hbm, bcol_hbm,
          *rest):
        wid = lax.axis_index("s") * NC + lax.axis_index("c")
        bufs = (rest[0:8], rest[8:16], rest[16:24])

        def issue(i, b):
            ci, ri, rc, rr, bv, s1, s2, s3 = bufs[b]
            ch = wid + i * NTILES

            @pl.when(ch < nchunks)
            def _():
                base = ch * CHUNK
                pltpu.sync_copy(eidx_hbm.at[0, pl.ds(base, CHUNK)], ci)
                pltpu.sync_copy(eidx_hbm.at[1, pl.ds(base, CHUNK)], ri)
                pltpu.async_copy(x_hbm.at[ci], rc, s1)
                pltpu.async_copy(x_hbm.at[ri], rr, s2)
                pltpu.async_copy(batch_hbm.at[ci], bv, s3)

        def drain(i, b):
            ci, ri, rc, rr, bv, s1, s2, s3 = bufs[b]
            ch = wid + i * NTILES

            @pl.when(ch < nchunks)
            def _():
                base = ch * CHUNK
                pltpu.make_async_copy(x_hbm.at[ci], rc, s1).wait()
                pltpu.make_async_copy(x_hbm.at[ri], rr, s2).wait()
                pltpu.make_async_copy(batch_hbm.at[ci], bv, s3).wait()
                pltpu.sync_copy(rc, gxc_hbm.at[pl.ds(base, CHUNK)])
                pltpu.sync_copy(rr, gxr_hbm.at[pl.ds(base, CHUNK)])
                pltpu.sync_copy(bv, bcol_hbm.at[pl.ds(base, CHUNK)])

        issue(0, 0)
        issue(1, 1)

        def body(i3, carry):
            i = 3 * i3
            issue(i + 2, 2)
            drain(i, 0)
            issue(i + 3, 0)
            drain(i + 1, 1)
            issue(i + 4, 1)
            drain(i + 2, 2)
            return carry

        lax.fori_loop(0, np3, body, 0)

    return k(x, edge_index, batch)


NP = 10240
NCHUNKS = NNZ // CHUNK
HE3R = 2560


def _sc_hconv_pass(table, hyper_edge3, gdim, sdim, stacked,
                   scalar_mode=None, nw=None, sched_dep=None):
    IB = 8
    stride = NS if stacked else NTILES
    cpt = IB * (-(-NCHUNKS // stride // IB))
    nbat = cpt // IB
    rpt = NP // NS
    nsub = CHUNK // 16

    out_type = [jax.ShapeDtypeStruct((NC, NP, F), jnp.float32)]
    if scalar_mode:
        out_type.append(jax.ShapeDtypeStruct((NP,), jnp.float32))

    scratch = [
        pltpu.VMEM((IB, CHUNK), jnp.int32),
        pltpu.VMEM((IB, CHUNK), jnp.int32),
        pltpu.VMEM((IB, CHUNK), jnp.int32),
        pltpu.VMEM((IB, CHUNK), jnp.int32),
        pltpu.VMEM((CHUNK,), jnp.int32),
        pltpu.VMEM((CHUNK,), jnp.int32),
        pltpu.VMEM((CHUNK, F), jnp.float32),
        pltpu.VMEM((CHUNK, F), jnp.float32),
        pltpu.SemaphoreType.DMA,
        pltpu.SemaphoreType.DMA,
        pltpu.VMEM_SHARED((NP, F), jnp.float32),
    ]
    if scalar_mode:
        scratch += [pltpu.VMEM_SHARED((NP,), jnp.float32),
                    pltpu.VMEM((CHUNK,), jnp.float32),
                    pltpu.VMEM((CHUNK,), jnp.float32),
                    pltpu.SemaphoreType.DMA,
                    pltpu.SemaphoreType.DMA]

    @functools.partial(pl.kernel, mesh=_mesh(), out_type=out_type,
                       scratch_types=scratch)
    def k(tab_hbm, he_hbm, zero_hbm, zero1_hbm, ones_hbm, nw_hbm, dep_hbm,
          *rest):
        if scalar_mode:
            acc_out, sc_out = rest[0], rest[1]
            rest = rest[2:]
        else:
            acc_out = rest[0]
            rest = rest[1:]
        gio_s = rest[0:2]
        si_s = rest[2:4]
        gim_s = rest[4:6]
        rows_s = rest[6:8]
        sem_s = rest[8:10]
        acc_sh = rest[10]
        if scalar_mode:
            sacc_sh = rest[11]
            wa_s = rest[12:14]
            wsem_s = rest[14:16]
            ones_v = wa_s[0]
        cid = lax.axis_index("c")
        sid = lax.axis_index("s")
        wid = sid * NC + cid
        goff = cid * N
        tbase = (sid if stacked else wid) * cpt

        tb = sid * rpt
        pltpu.sync_copy(zero_hbm.at[pl.ds(tb, rpt)], acc_sh.at[pl.ds(tb, rpt)])
        if scalar_mode == "cnt":
            pltpu.sync_copy(ones_hbm, ones_v)
        if scalar_mode:
            @pl.when((cid == 0) & (sid == 0))
            def _():
                pltpu.sync_copy(zero1_hbm, sacc_sh)
        plsc.subcore_barrier()

        def load_batch(b, bs):
            ch0 = tbase + b * IB

            @pl.when(ch0 < NCHUNKS)
            def _():
                pltpu.sync_copy(he_hbm.at[gdim, pl.ds(ch0, IB)], gio_s[bs])
                pltpu.sync_copy(he_hbm.at[sdim, pl.ds(ch0, IB)], si_s[bs])

        def issue(q, j, bs, rb):
            ch = tbase + q

            @pl.when(ch < NCHUNKS)
            def _():
                if stacked:
                    for u in range(nsub):
                        sl = pl.ds(u * 16, 16)
                        gim_s[rb][sl] = gio_s[bs][j, sl] + goff
                    gsrc = gim_s[rb]
                else:
                    gsrc = gio_s[bs].at[j]
                pltpu.async_copy(tab_hbm.at[gsrc], rows_s[rb], sem_s[rb])
                if scalar_mode == "dn":
                    @pl.when(cid == 0)
                    def _():
                        pltpu.async_copy(nw_hbm.at[gio_s[bs].at[j]],
                                         wa_s[rb], wsem_s[rb])

        def drain(q, j, bs, rb):
            ch = tbase + q

            @pl.when((q >= 0) & (ch < NCHUNKS))
            def _():
                gsrc = gim_s[rb] if stacked else gio_s[bs].at[j]
                pltpu.make_async_copy(tab_hbm.at[gsrc], rows_s[rb],
                                      sem_s[rb]).wait()
                pltpu.sync_copy(rows_s[rb], acc_sh.at[si_s[bs].at[j]],
                                add=True)
                if scalar_mode == "cnt":
                    @pl.when(cid == 0)
                    def _():
                        pltpu.sync_copy(ones_v, sacc_sh.at[si_s[bs].at[j]],
                                        add=True)
                elif scalar_mode == "dn":
                    @pl.when(cid == 0)
                    def _():
                        pltpu.make_async_copy(nw_hbm.at[gio_s[bs].at[j]],
                                              wa_s[rb], wsem_s[rb]).wait()
                        pltpu.sync_copy(wa_s[rb], sacc_sh.at[si_s[bs].at[j]],
                                        add=True)

        def pair_body(bp, carry):
            for half in (0, 1):
                b = 2 * bp + half
                load_batch(b, half)
                for j in range(IB):
                    q = b * IB + j
                    issue(q, j, half, j % 2)
                    if j > 0:
                        drain(q - 1, j - 1, half, (j - 1) % 2)
                    else:
                        drain(q - 1, IB - 1, 1 - half, (IB - 1) % 2)
            return carry

        npair = -(-nbat // 2)
        lax.fori_loop(0, npair, pair_body, 0)
        qlast = 2 * npair * IB - 1
        drain(qlast, IB - 1, 1, (IB - 1) % 2)
        plsc.subcore_barrier()
        pltpu.sync_copy(acc_sh.at[pl.ds(tb, rpt)],
                        acc_out.at[cid, pl.ds(tb, rpt)])
        if scalar_mode:
            @pl.when(cid == 0)
            def _():
                pltpu.sync_copy(sacc_sh.at[pl.ds(tb, rpt)],
                                sc_out.at[pl.ds(tb, rpt)])

    zeros = jnp.zeros((NP, F), jnp.float32)
    zeros1 = jnp.zeros((NP,), jnp.float32)
    ones = jnp.ones((CHUNK,), jnp.float32)
    if nw is None:
        nw = jnp.zeros((N,), jnp.float32)
    if sched_dep is None:
        sched_dep = ones
    res = k(table, hyper_edge3, zeros, zeros1, ones, nw, sched_dep)
    return res if scalar_mode else res[0]


NBLK = 2000
NNB = N // NBLK

_full = lambda shape: pl.BlockSpec(shape, lambda i: (0,) * len(shape))
_nrow = lambda w: pl.BlockSpec((NBLK, w), lambda i: (i, 0))


def _node_body(gemb_tab_ref, whl1a_ref, whl1b_ref, bhl1_ref, whl2_ref,
               bhl2_ref, x_ref, batch_ref, nw_ref):
    b = batch_ref[0, 0]
    onehot = (b[:, None] == lax.broadcasted_iota(jnp.int32, (1, G), 1)
              ).astype(jnp.float32)
    proto = jnp.dot(onehot, gemb_tab_ref[...],
                    preferred_element_type=jnp.float32)
    h = jnp.dot(x_ref[...], whl1a_ref[...], preferred_element_type=jnp.float32)
    h += jnp.dot(proto, whl1b_ref[...], preferred_element_type=jnp.float32)
    h = jnp.maximum(h + bhl1_ref[...], 0.0)
    nw = jnp.dot(h, whl2_ref[...], preferred_element_type=jnp.float32) \
        + bhl2_ref[...]
    nw_ref[...] = jax.nn.sigmoid(nw)


def _node_stage(x, graph_emb, batch, Whl1, bhl1, Whl2, bhl2):
    return pl.pallas_call(
        _node_body,
        grid=(NNB,),
        in_specs=[
            _full((G, F)), _full((F, F)), _full((F, F)), _full((1, F)),
            _full((F, 1)), _full((1, 1)),
            _nrow(F),
            pl.BlockSpec((1, 1, NBLK), lambda i: (i, 0, 0)),
        ],
        out_specs=_nrow(1),
        out_shape=jax.ShapeDtypeStruct((N, 1), jnp.float32),
    )(graph_emb, Whl1[:F], Whl1[F:], bhl1.reshape(1, -1), Whl2,
      bhl2.reshape(1, -1), x, batch.reshape(NNB, 1, NBLK))


_stk = pl.BlockSpec((2, NBLK, F), lambda i: (0, i, 0))
_scal = pl.BlockSpec((NBLK, 1), lambda i: (i, 0))


def _prep1_body(wa_ref, wb_ref, b_ref, gxc_ref, gxr_ref, out_ref):
    Xl = jnp.dot(gxc_ref[...], wa_ref[...], preferred_element_type=jnp.float32)
    Xl += jnp.dot(gxr_ref[...], wb_ref[...], preferred_element_type=jnp.float32)
    Xl += b_ref[...]
    out_ref[...] = jnp.stack([Xl[:, :F], Xl[:, F:]], axis=0)


def _prep1(gxc, gxr, Whc1, bhc1):
    return pl.pallas_call(
        _prep1_body,
        grid=(NNB,),
        in_specs=[_full((F, 2 * F)), _full((F, 2 * F)), _full((1, 2 * F)),
                  _nrow(F), _nrow(F)],
        out_specs=_stk,
        out_shape=jax.ShapeDtypeStruct((2, N, F), jnp.float32),
    )(Whc1[:F], Whc1[F:], bhc1.reshape(1, -1), gxc, gxr)


def _prep2_body(s1_ref, cntp_ref, nw_ref, out_ref, scale_ref):
    cnt = cntp_ref[...][:, 0]
    nw = nw_ref[...][:, 0]
    s = jnp.where(cnt > 0, nw / cnt, 0.0)
    out_ref[...] = s[None, :, None] * s1_ref[...]
    scale_ref[...] = s[:, None]


def _prep2(S1, cntp, nw):
    return pl.pallas_call(
        _prep2_body,
        grid=(NNB,),
        in_specs=[_stk, _scal, _nrow(1)],
        out_specs=[_stk, _nrow(1)],
        out_shape=[jax.ShapeDtypeStruct((2, N, F), jnp.float32),
                   jax.ShapeDtypeStruct((N, 1), jnp.float32)],
    )(S1, cntp, nw)


def _prep3_body(whc2a_ref, whc2b_ref, b_ref, t1_ref, dnp_ref,
                out_ref, dinv_ref):
    dn = dnp_ref[...][:, 0]
    dinv = jnp.where(dn > 0, 1.0 / dn, 0.0)
    era = jax.nn.sigmoid(dinv[:, None] * t1_ref[0])
    erb = jax.nn.sigmoid(dinv[:, None] * t1_ref[1])
    Xl2 = jnp.dot(era, whc2a_ref[...], preferred_element_type=jnp.float32)
    Xl2 += jnp.dot(erb, whc2b_ref[...], preferred_element_type=jnp.float32)
    out_ref[...] = Xl2 + b_ref[...]
    dinv_ref[...] = dinv[:, None]


def _prep3(T1, dnp, Whc2, bhc2):
    return pl.pallas_call(
        _prep3_body,
        grid=(NNB,),
        in_specs=[_full((F, F)), _full((F, F)), _full((1, F)), _stk, _scal],
        out_specs=[_nrow(F), _nrow(1)],
        out_shape=[jax.ShapeDtypeStruct((N, F), jnp.float32),
                   jax.ShapeDtypeStruct((N, 1), jnp.float32)],
    )(Whc2[:F], Whc2[F:], bhc2.reshape(1, -1), T1, dnp)


def _prep4_body(s2_ref, scale_ref, out_ref):
    out_ref[...] = scale_ref[...] * (s2_ref[0] + s2_ref[1])


def _prep4(S2, scale):
    return pl.pallas_call(
        _prep4_body,
        grid=(NNB,),
        in_specs=[_stk, _nrow(1)],
        out_specs=_nrow(F),
        out_shape=jax.ShapeDtypeStruct((N, F), jnp.float32),
    )(S2, scale)


def _prep5_body(t2_ref, dinv_ref, out_ref):
    out_ref[...] = jax.nn.sigmoid(dinv_ref[...] * (t2_ref[0] + t2_ref[1]))


def _prep5(T2, dinv):
    return pl.pallas_call(
        _prep5_body,
        grid=(NNB,),
        in_specs=[_stk, _nrow(1)],
        out_specs=_nrow(F),
        out_shape=jax.ShapeDtypeStruct((N, F), jnp.float32),
    )(T2, dinv)


EBLK = 2000
NEB = E // EBLK
NSH = N // EBLK


def _edge_heavy_body(gemb_tab_ref, wl1a_ref, wl1b_ref, wl1c_ref, bl1_ref,
                     wl2_ref, bl2_ref, gxc_ref, gxr_ref, bcol_ref, out_ref):
    bcol = bcol_ref[0, 0]
    onehot = (bcol[:, None] == lax.broadcasted_iota(jnp.int32, (1, G), 1)
              ).astype(jnp.float32)
    gemb = jnp.dot(onehot, gemb_tab_ref[...],
                   preferred_element_type=jnp.float32)
    h1 = jnp.dot(gxc_ref[...], wl1a_ref[...], preferred_element_type=jnp.float32)
    h1 += jnp.dot(gxr_ref[...], wl1b_ref[...], preferred_element_type=jnp.float32)
    h1 += jnp.dot(gemb, wl1c_ref[...], preferred_element_type=jnp.float32)
    h1 = jnp.maximum(h1 + bl1_ref[...], 0.0)
    out_ref[...] = jnp.maximum(
        jnp.dot(h1, wl2_ref[...], preferred_element_type=jnp.float32)
        + bl2_ref[...], 0.0)


def _edge_heavy(gxc, gxr, bcol, graph_emb, Wl1, bl1, Wl2, bl2):
    return pl.pallas_call(
        _edge_heavy_body,
        grid=(NEB,),
        in_specs=[
            _full((G, F)),
            _full((F, 4 * F)), _full((F, 4 * F)), _full((F, 4 * F)),
            _full((1, 4 * F)),
            _full((4 * F, F)), _full((1, F)),
            pl.BlockSpec((EBLK, F), lambda i: (i, 0)),
            pl.BlockSpec((EBLK, F), lambda i: (i, 0)),
            pl.BlockSpec((1, 1, EBLK), lambda i: (i, 0, 0)),
        ],
        out_specs=pl.BlockSpec((EBLK, F), lambda i: (i, 0)),
        out_shape=jax.ShapeDtypeStruct((E, F), jnp.float32),
    )(graph_emb, Wl1[:F], Wl1[F:2 * F], Wl1[2 * F:], bl1.reshape(1, -1),
      Wl2, bl2.reshape(1, -1), gxc, gxr, bcol.reshape(NEB, 1, EBLK))


def _light_core(gemb_tab_ref, wc1a_ref, wc1b_ref, bc1_ref, wc2_ref, bc2_ref,
                attn_ref, xij2_ref, bcol_ref, sh, out_ref):
    bcol = bcol_ref[0, 0]
    onehot = (bcol[:, None] == lax.broadcasted_iota(jnp.int32, (1, G), 1)
              ).astype(jnp.float32)
    gemb = jnp.dot(onehot, gemb_tab_ref[...],
                   preferred_element_type=jnp.float32)
    s = attn_ref[0, 0] * xij2_ref[...] + attn_ref[0, 1] * sh
    z = jnp.dot(s, wc1a_ref[...], preferred_element_type=jnp.float32)
    z += jnp.dot(gemb, wc1b_ref[...], preferred_element_type=jnp.float32)
    z = jnp.maximum(z + bc1_ref[...], 0.0)
    o = jnp.dot(z, wc2_ref[...], preferred_element_type=jnp.float32) \
        + bc2_ref[...]
    out_ref[...] = jax.nn.sigmoid(o)


def _light_main_body(gemb_tab_ref, wc1a_ref, wc1b_ref, bc1_ref,
                     wc2_ref, bc2_ref, attn_ref, xij2_ref, bcol_ref, out_ref):
    _light_core(gemb_tab_ref, wc1a_ref, wc1b_ref, bc1_ref, wc2_ref, bc2_ref,
                attn_ref, xij2_ref, bcol_ref, 0.5, out_ref)


def _light_pref_body(gemb_tab_ref, wc1a_ref, wc1b_ref, bc1_ref,
                     wc2_ref, bc2_ref, attn_ref, xij2_ref, bcol_ref,
                     t2_ref, dinv_ref, out_ref):
    sh = jax.nn.sigmoid(dinv_ref[...] * (t2_ref[0] + t2_ref[1]))
    _light_core(gemb_tab_ref, wc1a_ref, wc1b_ref, bc1_ref, wc2_ref, bc2_ref,
                attn_ref, xij2_ref, bcol_ref, sh, out_ref)


_LIGHT_W = lambda: [_full((G, F)), _full((F, F)), _full((F, F)),
                    _full((1, F)), _full((F, 1)), _full((1, 1)),
                    _full((1, 2))]


def _light_wargs(graph_emb, Wc1, bc1, Wc2, bc2, attn):
    return (graph_emb, Wc1[:F], Wc1[F:], bc1.reshape(1, -1),
            Wc2, bc2.reshape(1, -1), attn.reshape(1, 2))


def _edge_light_main(xij2, bcol, w_args):
    return pl.pallas_call(
        _light_main_body,
        grid=(NEB - NSH,),
        in_specs=_LIGHT_W() + [
            pl.BlockSpec((EBLK, F), lambda i: (i + NSH, 0)),
            pl.BlockSpec((1, 1, EBLK), lambda i: (i + NSH, 0, 0)),
        ],
        out_specs=pl.BlockSpec((EBLK, 1), lambda i: (i, 0)),
        out_shape=jax.ShapeDtypeStruct((E - N, 1), jnp.float32),
    )(*w_args, xij2, bcol.reshape(NEB, 1, EBLK))


def _edge_light_prefix(xij2, bcol, T2, dinv, w_args):
    return pl.pallas_call(
        _light_pref_body,
        grid=(NSH,),
        in_specs=_LIGHT_W() + [
            pl.BlockSpec((EBLK, F), lambda i: (i, 0)),
            pl.BlockSpec((1, 1, EBLK), lambda i: (i, 0, 0)),
            pl.BlockSpec((2, EBLK, F), lambda i: (0, i, 0)),
            pl.BlockSpec((EBLK, 1), lambda i: (i, 0)),
        ],
        out_specs=pl.BlockSpec((EBLK, 1), lambda i: (i, 0)),
        out_shape=jax.ShapeDtypeStruct((N, 1), jnp.float32),
    )(*w_args, xij2, bcol.reshape(NEB, 1, EBLK), T2, dinv)


def kernel(x, graph_emb, edge_index, edge_type, batch, hyper_edge, attn,
           Whl1, bhl1, Whl2, bhl2, Whc1, bhc1, Whc2, bhc2,
           Wl1, bl1, Wl2, bl2, Wc1, bc1, Wc2, bc2):
    nw = _node_stage(x, graph_emb, batch, Whl1, bhl1, Whl2, bhl2)

    gxc, gxr, bcol = _sc_edge_gather(x, edge_index, batch)

    xij2 = _edge_heavy(gxc, gxr, bcol, graph_emb, Wl1, bl1, Wl2, bl2)

    he3 = jnp.pad(hyper_edge.reshape(2, NCHUNKS, CHUNK),
                  ((0, 0), (0, HE3R - NCHUNKS), (0, 0)))
    Xl1e = _prep1(gxc, gxr, Whc1, bhc1)
    S1, cntp = _sc_hconv_pass(Xl1e.reshape(2 * N, F), he3, 0, 1,
                              stacked=True, scalar_mode="cnt")
    a1e, scale = _prep2(S1, cntp.reshape(NP, 1), nw)
    T1, dnp = _sc_hconv_pass(a1e.reshape(2 * N, F), he3, 1, 0,
                             stacked=True, scalar_mode="dn", nw=nw[:, 0],
                             sched_dep=xij2)
    w_args = _light_wargs(graph_emb, Wc1, bc1, Wc2, bc2, attn)
    sij_main = _edge_light_main(xij2, bcol, w_args)

    Xl2, dinv = _prep3(T1, dnp.reshape(NP, 1), Whc2, bhc2)
    S2 = _sc_hconv_pass(Xl2, he3, 0, 1, stacked=False,
                        sched_dep=sij_main)
    a2 = _prep4(S2, scale)
    T2 = _sc_hconv_pass(a2, he3, 1, 0, stacked=False)

    sij_pref = _edge_light_prefix(xij2, bcol, T2, dinv, w_args)
    sij = jnp.concatenate([sij_pref, sij_main], axis=0)
    return (edge_index, edge_type, sij)


# speedup vs baseline: 1.4431x; 1.0364x over previous
import functools

import jax
import jax.numpy as jnp
from jax import lax
from jax.experimental import pallas as pl
from jax.experimental.pallas import tpu as pltpu
from jax.experimental.pallas import tpu_sc as plsc

N = 10000
E = 160000
NNZ = 320000
G = 64
F = 128

NC = 2
NS = 16
NTILES = NC * NS
CHUNK = 128
CEXT = 144
CH2 = 64

_mesh = lambda: plsc.VectorSubcoreMesh(core_axis_name="c", subcore_axis_name="s")


def _sc_edge_gather(x, edge_index3, batch):
    IB = 8
    nch = E // CHUNK
    cpt = 40
    nbat = cpt // IB

    @functools.partial(
        pl.kernel,
        mesh=_mesh(),
        out_type=[
            jax.ShapeDtypeStruct((E, F), jnp.float32),
            jax.ShapeDtypeStruct((E, F), jnp.float32),
            jax.ShapeDtypeStruct((E,), jnp.int32),
        ],
        scratch_types=[
            pltpu.VMEM((IB, CHUNK), jnp.int32),
            pltpu.VMEM((IB, CHUNK), jnp.int32),
            pltpu.VMEM((IB, CHUNK), jnp.int32),
            pltpu.VMEM((IB, CHUNK), jnp.int32),
            pltpu.VMEM((CHUNK, F), jnp.float32),
            pltpu.VMEM((CHUNK, F), jnp.float32),
            pltpu.VMEM((CHUNK, F), jnp.float32),
            pltpu.VMEM((CHUNK, F), jnp.float32),
            pltpu.VMEM((CHUNK,), jnp.int32),
            pltpu.VMEM((CHUNK,), jnp.int32),
            pltpu.SemaphoreType.DMA, pltpu.SemaphoreType.DMA,
            pltpu.SemaphoreType.DMA, pltpu.SemaphoreType.DMA,
            pltpu.SemaphoreType.DMA, pltpu.SemaphoreType.DMA,
        ],
    )
    def k(x_hbm, eidx_hbm, batch_hbm, gxc_hbm, gxr_hbm, bcol_hbm, *rest):
        ci_s = rest[0:2]
        ri_s = rest[2:4]
        rc_s = rest[4:6]
        rr_s = rest[6:8]
        bv_s = rest[8:10]
        s1_s = rest[10:12]
        s2_s = rest[12:14]
        s3_s = rest[14:16]
        wid = lax.axis_index("s") * NC + lax.axis_index("c")
        tbase = wid * cpt

        def load_batch(b, bs):
            ch0 = tbase + b * IB

            @pl.when(ch0 < nch)
            def _():
                pltpu.sync_copy(eidx_hbm.at[0, pl.ds(ch0, IB)], ci_s[bs])
                pltpu.sync_copy(eidx_hbm.at[1, pl.ds(ch0, IB)], ri_s[bs])

        def issue(q, j, bs, rb):
            @pl.when(tbase + q < nch)
            def _():
                pltpu.async_copy(x_hbm.at[ci_s[bs].at[j]], rc_s[rb], s1_s[rb])
                pltpu.async_copy(x_hbm.at[ri_s[bs].at[j]], rr_s[rb], s2_s[rb])
                pltpu.async_copy(batch_hbm.at[ci_s[bs].at[j]], bv_s[rb],
                                 s3_s[rb])

        def drain(q, j, bs, rb):
            @pl.when((q >= 0) & (tbase + q < nch))
            def _():
                base = (tbase + q) * CHUNK
                pltpu.make_async_copy(x_hbm.at[ci_s[bs].at[j]], rc_s[rb],
                                      s1_s[rb]).wait()
                pltpu.make_async_copy(x_hbm.at[ri_s[bs].at[j]], rr_s[rb],
                                      s2_s[rb]).wait()
                pltpu.make_async_copy(batch_hbm.at[ci_s[bs].at[j]], bv_s[rb],
                                      s3_s[rb]).wait()
                pltpu.sync_copy(rc_s[rb], gxc_hbm.at[pl.ds(base, CHUNK)])
                pltpu.sync_copy(rr_s[rb], gxr_hbm.at[pl.ds(base, CHUNK)])
                pltpu.sync_copy(bv_s[rb], bcol_hbm.at[pl.ds(base, CHUNK)])

        def pair_body(bp, carry):
            for half in (0, 1):
                b = 2 * bp + half
                load_batch(b, half)
                for j in range(IB):
                    q = b * IB + j
                    issue(q, j, half, j % 2)
                    if j > 0:
                        drain(q - 1, j - 1, half, (j - 1) % 2)
                    else:
                        drain(q - 1, IB - 1, 1 - half, (IB - 1) % 2)
            return carry

        npair = nbat // 2
        lax.fori_loop(0, npair, pair_body, 0)
        qlast = nbat * IB - 1
        drain(qlast, IB - 1, 1, (IB - 1) % 2)

    return k(x, edge_index3, batch)


NP = 10240
NCHUNKS = NNZ // CHUNK
HE3R = 2560


def _sc_hconv_pass(table, hyper_edge3, gdim, sdim, stacked,
                   scalar_mode=None, nw=None, sched_dep=None):
    IB = 8
    stride = NS if stacked else NTILES
    cpt = IB * (-(-NCHUNKS // stride // IB))
    nbat = cpt // IB
    rpt = NP // NS
    nsub = CHUNK // 16

    out_type = [jax.ShapeDtypeStruct((NC, NP, F), jnp.float32)]
    if scalar_mode:
        out_type.append(jax.ShapeDtypeStruct((NP,), jnp.float32))

    scratch = [
        pltpu.VMEM((IB, CHUNK), jnp.int32),
        pltpu.VMEM((IB, CHUNK), jnp.int32),
        pltpu.VMEM((IB, CHUNK), jnp.int32),
        pltpu.VMEM((IB, CHUNK), jnp.int32),
        pltpu.VMEM((CHUNK,), jnp.int32),
        pltpu.VMEM((CHUNK,), jnp.int32),
        pltpu.VMEM((CHUNK, F), jnp.float32),
        pltpu.VMEM((CHUNK, F), jnp.float32),
        pltpu.SemaphoreType.DMA,
        pltpu.SemaphoreType.DMA,
        pltpu.VMEM_SHARED((NP, F), jnp.float32),
    ]
    if scalar_mode:
        scratch += [pltpu.VMEM_SHARED((NP,), jnp.float32),
                    pltpu.VMEM((CHUNK,), jnp.float32),
                    pltpu.VMEM((CHUNK,), jnp.float32),
                    pltpu.SemaphoreType.DMA,
                    pltpu.SemaphoreType.DMA]

    @functools.partial(pl.kernel, mesh=_mesh(), out_type=out_type,
                       scratch_types=scratch)
    def k(tab_hbm, he_hbm, zero_hbm, zero1_hbm, ones_hbm, nw_hbm, dep_hbm,
          *rest):
        if scalar_mode:
            acc_out, sc_out = rest[0], rest[1]
            rest = rest[2:]
        else:
            acc_out = rest[0]
            rest = rest[1:]
        gio_s = rest[0:2]
        si_s = rest[2:4]
        gim_s = rest[4:6]
        rows_s = rest[6:8]
        sem_s = rest[8:10]
        acc_sh = rest[10]
        if scalar_mode:
            sacc_sh = rest[11]
            wa_s = rest[12:14]
            wsem_s = rest[14:16]
            ones_v = wa_s[0]
        cid = lax.axis_index("c")
        sid = lax.axis_index("s")
        wid = sid * NC + cid
        goff = cid * N
        tbase = (sid if stacked else wid) * cpt

        tb = sid * rpt
        pltpu.sync_copy(zero_hbm.at[pl.ds(tb, rpt)], acc_sh.at[pl.ds(tb, rpt)])
        if scalar_mode == "cnt":
            pltpu.sync_copy(ones_hbm, ones_v)
        if scalar_mode:
            @pl.when((cid == 0) & (sid == 0))
            def _():
                pltpu.sync_copy(zero1_hbm, sacc_sh)
        plsc.subcore_barrier()

        def load_batch(b, bs):
            ch0 = tbase + b * IB

            @pl.when(ch0 < NCHUNKS)
            def _():
                pltpu.sync_copy(he_hbm.at[gdim, pl.ds(ch0, IB)], gio_s[bs])
                pltpu.sync_copy(he_hbm.at[sdim, pl.ds(ch0, IB)], si_s[bs])

        def issue(q, j, bs, rb):
            ch = tbase + q

            @pl.when(ch < NCHUNKS)
            def _():
                if stacked:
                    for u in range(nsub):
                        sl = pl.ds(u * 16, 16)
                        gim_s[rb][sl] = gio_s[bs][j, sl] + goff
                    gsrc = gim_s[rb]
                else:
                    gsrc = gio_s[bs].at[j]
                pltpu.async_copy(tab_hbm.at[gsrc], rows_s[rb], sem_s[rb])
                if scalar_mode == "dn":
                    @pl.when(cid == 0)
                    def _():
                        pltpu.async_copy(nw_hbm.at[gio_s[bs].at[j]],
                                         wa_s[rb], wsem_s[rb])

        def drain(q, j, bs, rb):
            ch = tbase + q

            @pl.when((q >= 0) & (ch < NCHUNKS))
            def _():
                gsrc = gim_s[rb] if stacked else gio_s[bs].at[j]
                pltpu.make_async_copy(tab_hbm.at[gsrc], rows_s[rb],
                                      sem_s[rb]).wait()
                pltpu.sync_copy(rows_s[rb], acc_sh.at[si_s[bs].at[j]],
                                add=True)
                if scalar_mode == "cnt":
                    @pl.when(cid == 0)
                    def _():
                        pltpu.sync_copy(ones_v, sacc_sh.at[si_s[bs].at[j]],
                                        add=True)
                elif scalar_mode == "dn":
                    @pl.when(cid == 0)
                    def _():
                        pltpu.make_async_copy(nw_hbm.at[gio_s[bs].at[j]],
                                              wa_s[rb], wsem_s[rb]).wait()
                        pltpu.sync_copy(wa_s[rb], sacc_sh.at[si_s[bs].at[j]],
                                        add=True)

        def pair_body(bp, carry):
            for half in (0, 1):
                b = 2 * bp + half
                load_batch(b, half)
                for j in range(IB):
                    q = b * IB + j
                    issue(q, j, half, j % 2)
                    if j > 0:
                        drain(q - 1, j - 1, half, (j - 1) % 2)
                    else:
                        drain(q - 1, IB - 1, 1 - half, (IB - 1) % 2)
            return carry

        npair = -(-nbat // 2)
        lax.fori_loop(0, npair, pair_body, 0)
        qlast = 2 * npair * IB - 1
        drain(qlast, IB - 1, 1, (IB - 1) % 2)
        plsc.subcore_barrier()
        pltpu.sync_copy(acc_sh.at[pl.ds(tb, rpt)],
                        acc_out.at[cid, pl.ds(tb, rpt)])
        if scalar_mode:
            @pl.when(cid == 0)
            def _():
                pltpu.sync_copy(sacc_sh.at[pl.ds(tb, rpt)],
                                sc_out.at[pl.ds(tb, rpt)])

    zeros = jnp.zeros((NP, F), jnp.float32)
    zeros1 = jnp.zeros((NP,), jnp.float32)
    ones = jnp.ones((CHUNK,), jnp.float32)
    if nw is None:
        nw = jnp.zeros((N,), jnp.float32)
    if sched_dep is None:
        sched_dep = ones
    res = k(table, hyper_edge3, zeros, zeros1, ones, nw, sched_dep)
    return res if scalar_mode else res[0]


NBLK = 2000
NNB = N // NBLK

_full = lambda shape: pl.BlockSpec(shape, lambda i: (0,) * len(shape))
_nrow = lambda w: pl.BlockSpec((NBLK, w), lambda i: (i, 0))


def _node_body(gemb_tab_ref, whl1a_ref, whl1b_ref, bhl1_ref, whl2_ref,
               bhl2_ref, x_ref, batch_ref, nw_ref):
    b = batch_ref[0, 0]
    onehot = (b[:, None] == lax.broadcasted_iota(jnp.int32, (1, G), 1)
              ).astype(jnp.float32)
    proto = jnp.dot(onehot, gemb_tab_ref[...],
                    preferred_element_type=jnp.float32)
    h = jnp.dot(x_ref[...], whl1a_ref[...], preferred_element_type=jnp.float32)
    h += jnp.dot(proto, whl1b_ref[...], preferred_element_type=jnp.float32)
    h = jnp.maximum(h + bhl1_ref[...], 0.0)
    nw = jnp.dot(h, whl2_ref[...], preferred_element_type=jnp.float32) \
        + bhl2_ref[...]
    nw_ref[...] = jax.nn.sigmoid(nw)


def _node_stage(x, graph_emb, batch, Whl1, bhl1, Whl2, bhl2):
    return pl.pallas_call(
        _node_body,
        grid=(NNB,),
        in_specs=[
            _full((G, F)), _full((F, F)), _full((F, F)), _full((1, F)),
            _full((F, 1)), _full((1, 1)),
            _nrow(F),
            pl.BlockSpec((1, 1, NBLK), lambda i: (i, 0, 0)),
        ],
        out_specs=_nrow(1),
        out_shape=jax.ShapeDtypeStruct((N, 1), jnp.float32),
    )(graph_emb, Whl1[:F], Whl1[F:], bhl1.reshape(1, -1), Whl2,
      bhl2.reshape(1, -1), x, batch.reshape(NNB, 1, NBLK))


_stk = pl.BlockSpec((2, NBLK, F), lambda i: (0, i, 0))
_scal = pl.BlockSpec((NBLK, 1), lambda i: (i, 0))


def _prep1_body(wa_ref, wb_ref, b_ref, gxc_ref, gxr_ref, out_ref):
    Xl = jnp.dot(gxc_ref[...], wa_ref[...], preferred_element_type=jnp.float32)
    Xl += jnp.dot(gxr_ref[...], wb_ref[...], preferred_element_type=jnp.float32)
    Xl += b_ref[...]
    out_ref[...] = jnp.stack([Xl[:, :F], Xl[:, F:]], axis=0)


def _prep1(gxc, gxr, Whc1, bhc1):
    return pl.pallas_call(
        _prep1_body,
        grid=(NNB,),
        in_specs=[_full((F, 2 * F)), _full((F, 2 * F)), _full((1, 2 * F)),
                  _nrow(F), _nrow(F)],
        out_specs=_stk,
        out_shape=jax.ShapeDtypeStruct((2, N, F), jnp.float32),
    )(Whc1[:F], Whc1[F:], bhc1.reshape(1, -1), gxc, gxr)


def _prep2_body(s1_ref, cntp_ref, nw_ref, out_ref, scale_ref):
    cnt = cntp_ref[...][:, 0]
    nw = nw_ref[...][:, 0]
    s = jnp.where(cnt > 0, nw / cnt, 0.0)
    out_ref[...] = s[None, :, None] * s1_ref[...]
    scale_ref[...] = s[:, None]


def _prep2(S1, cntp, nw):
    return pl.pallas_call(
        _prep2_body,
        grid=(NNB,),
        in_specs=[_stk, _scal, _nrow(1)],
        out_specs=[_stk, _nrow(1)],
        out_shape=[jax.ShapeDtypeStruct((2, N, F), jnp.float32),
                   jax.ShapeDtypeStruct((N, 1), jnp.float32)],
    )(S1, cntp, nw)


def _prep3_body(whc2a_ref, whc2b_ref, b_ref, t1_ref, dnp_ref,
                out_ref, dinv_ref):
    dn = dnp_ref[...][:, 0]
    dinv = jnp.where(dn > 0, 1.0 / dn, 0.0)
    era = jax.nn.sigmoid(dinv[:, None] * t1_ref[0])
    erb = jax.nn.sigmoid(dinv[:, None] * t1_ref[1])
    Xl2 = jnp.dot(era, whc2a_ref[...], preferred_element_type=jnp.float32)
    Xl2 += jnp.dot(erb, whc2b_ref[...], preferred_element_type=jnp.float32)
    out_ref[...] = Xl2 + b_ref[...]
    dinv_ref[...] = dinv[:, None]


def _prep3(T1, dnp, Whc2, bhc2):
    return pl.pallas_call(
        _prep3_body,
        grid=(NNB,),
        in_specs=[_full((F, F)), _full((F, F)), _full((1, F)), _stk, _scal],
        out_specs=[_nrow(F), _nrow(1)],
        out_shape=[jax.ShapeDtypeStruct((N, F), jnp.float32),
                   jax.ShapeDtypeStruct((N, 1), jnp.float32)],
    )(Whc2[:F], Whc2[F:], bhc2.reshape(1, -1), T1, dnp)


def _prep4_body(s2_ref, scale_ref, out_ref):
    out_ref[...] = scale_ref[...] * (s2_ref[0] + s2_ref[1])


def _prep4(S2, scale):
    return pl.pallas_call(
        _prep4_body,
        grid=(NNB,),
        in_specs=[_stk, _nrow(1)],
        out_specs=_nrow(F),
        out_shape=jax.ShapeDtypeStruct((N, F), jnp.float32),
    )(S2, scale)


def _prep5_body(t2_ref, dinv_ref, out_ref):
    out_ref[...] = jax.nn.sigmoid(dinv_ref[...] * (t2_ref[0] + t2_ref[1]))


def _prep5(T2, dinv):
    return pl.pallas_call(
        _prep5_body,
        grid=(NNB,),
        in_specs=[_stk, _nrow(1)],
        out_specs=_nrow(F),
        out_shape=jax.ShapeDtypeStruct((N, F), jnp.float32),
    )(T2, dinv)


EBLK = 2000
NEB = E // EBLK
NSH = N // EBLK


def _edge_heavy_body(gemb_tab_ref, wl1a_ref, wl1b_ref, wl1c_ref, bl1_ref,
                     wl2_ref, bl2_ref, gxc_ref, gxr_ref, bcol_ref, out_ref):
    bcol = bcol_ref[0, 0]
    onehot = (bcol[:, None] == lax.broadcasted_iota(jnp.int32, (1, G), 1)
              ).astype(jnp.float32)
    gemb = jnp.dot(onehot, gemb_tab_ref[...],
                   preferred_element_type=jnp.float32)
    h1 = jnp.dot(gxc_ref[...], wl1a_ref[...], preferred_element_type=jnp.float32)
    h1 += jnp.dot(gxr_ref[...], wl1b_ref[...], preferred_element_type=jnp.float32)
    h1 += jnp.dot(gemb, wl1c_ref[...], preferred_element_type=jnp.float32)
    h1 = jnp.maximum(h1 + bl1_ref[...], 0.0)
    out_ref[...] = jnp.maximum(
        jnp.dot(h1, wl2_ref[...], preferred_element_type=jnp.float32)
        + bl2_ref[...], 0.0)


def _edge_heavy(gxc, gxr, bcol, graph_emb, Wl1, bl1, Wl2, bl2):
    return pl.pallas_call(
        _edge_heavy_body,
        grid=(NEB,),
        in_specs=[
            _full((G, F)),
            _full((F, 4 * F)), _full((F, 4 * F)), _full((F, 4 * F)),
            _full((1, 4 * F)),
            _full((4 * F, F)), _full((1, F)),
            pl.BlockSpec((EBLK, F), lambda i: (i, 0)),
            pl.BlockSpec((EBLK, F), lambda i: (i, 0)),
            pl.BlockSpec((1, 1, EBLK), lambda i: (i, 0, 0)),
        ],
        out_specs=pl.BlockSpec((EBLK, F), lambda i: (i, 0)),
        out_shape=jax.ShapeDtypeStruct((E, F), jnp.float32),
    )(graph_emb, Wl1[:F], Wl1[F:2 * F], Wl1[2 * F:], bl1.reshape(1, -1),
      Wl2, bl2.reshape(1, -1), gxc, gxr, bcol.reshape(NEB, 1, EBLK))


def _light_core(gemb_tab_ref, wc1a_ref, wc1b_ref, bc1_ref, wc2_ref, bc2_ref,
                attn_ref, xij2_ref, bcol_ref, sh, out_ref):
    bcol = bcol_ref[0, 0]
    onehot = (bcol[:, None] == lax.broadcasted_iota(jnp.int32, (1, G), 1)
              ).astype(jnp.float32)
    gemb = jnp.dot(onehot, gemb_tab_ref[...],
                   preferred_element_type=jnp.float32)
    s = attn_ref[0, 0] * xij2_ref[...] + attn_ref[0, 1] * sh
    z = jnp.dot(s, wc1a_ref[...], preferred_element_type=jnp.float32)
    z += jnp.dot(gemb, wc1b_ref[...], preferred_element_type=jnp.float32)
    z = jnp.maximum(z + bc1_ref[...], 0.0)
    o = jnp.dot(z, wc2_ref[...], preferred_element_type=jnp.float32) \
        + bc2_ref[...]
    out_ref[...] = jax.nn.sigmoid(o)


def _light_main_body(gemb_tab_ref, wc1a_ref, wc1b_ref, bc1_ref,
                     wc2_ref, bc2_ref, attn_ref, xij2_ref, bcol_ref, out_ref):
    _light_core(gemb_tab_ref, wc1a_ref, wc1b_ref, bc1_ref, wc2_ref, bc2_ref,
                attn_ref, xij2_ref, bcol_ref, 0.5, out_ref)


def _light_pref_body(gemb_tab_ref, wc1a_ref, wc1b_ref, bc1_ref,
                     wc2_ref, bc2_ref, attn_ref, xij2_ref, bcol_ref,
                     t2_ref, dinv_ref, out_ref):
    sh = jax.nn.sigmoid(dinv_ref[...] * (t2_ref[0] + t2_ref[1]))
    _light_core(gemb_tab_ref, wc1a_ref, wc1b_ref, bc1_ref, wc2_ref, bc2_ref,
                attn_ref, xij2_ref, bcol_ref, sh, out_ref)


_LIGHT_W = lambda: [_full((G, F)), _full((F, F)), _full((F, F)),
                    _full((1, F)), _full((F, 1)), _full((1, 1)),
                    _full((1, 2))]


def _light_wargs(graph_emb, Wc1, bc1, Wc2, bc2, attn):
    return (graph_emb, Wc1[:F], Wc1[F:], bc1.reshape(1, -1),
            Wc2, bc2.reshape(1, -1), attn.reshape(1, 2))


def _edge_light_main(xij2, bcol, w_args):
    return pl.pallas_call(
        _light_main_body,
        grid=(NEB - NSH,),
        in_specs=_LIGHT_W() + [
            pl.BlockSpec((EBLK, F), lambda i: (i + NSH, 0)),
            pl.BlockSpec((1, 1, EBLK), lambda i: (i + NSH, 0, 0)),
        ],
        out_specs=pl.BlockSpec((EBLK, 1), lambda i: (i, 0)),
        out_shape=jax.ShapeDtypeStruct((E - N, 1), jnp.float32),
    )(*w_args, xij2, bcol.reshape(NEB, 1, EBLK))


def _edge_light_prefix(xij2, bcol, T2, dinv, w_args):
    return pl.pallas_call(
        _light_pref_body,
        grid=(NSH,),
        in_specs=_LIGHT_W() + [
            pl.BlockSpec((EBLK, F), lambda i: (i, 0)),
            pl.BlockSpec((1, 1, EBLK), lambda i: (i, 0, 0)),
            pl.BlockSpec((2, EBLK, F), lambda i: (0, i, 0)),
            pl.BlockSpec((EBLK, 1), lambda i: (i, 0)),
        ],
        out_specs=pl.BlockSpec((EBLK, 1), lambda i: (i, 0)),
        out_shape=jax.ShapeDtypeStruct((N, 1), jnp.float32),
    )(*w_args, xij2, bcol.reshape(NEB, 1, EBLK), T2, dinv)


def kernel(x, graph_emb, edge_index, edge_type, batch, hyper_edge, attn,
           Whl1, bhl1, Whl2, bhl2, Whc1, bhc1, Whc2, bhc2,
           Wl1, bl1, Wl2, bl2, Wc1, bc1, Wc2, bc2):
    nw = _node_stage(x, graph_emb, batch, Whl1, bhl1, Whl2, bhl2)

    eidx3 = jnp.pad(edge_index.reshape(2, E // CHUNK, CHUNK),
                    ((0, 0), (0, 1280 - E // CHUNK), (0, 0)))
    gxc, gxr, bcol = _sc_edge_gather(x, eidx3, batch)

    xij2 = _edge_heavy(gxc, gxr, bcol, graph_emb, Wl1, bl1, Wl2, bl2)

    he3 = jnp.pad(hyper_edge.reshape(2, NCHUNKS, CHUNK),
                  ((0, 0), (0, HE3R - NCHUNKS), (0, 0)))
    Xl1e = _prep1(gxc, gxr, Whc1, bhc1)
    S1, cntp = _sc_hconv_pass(Xl1e.reshape(2 * N, F), he3, 0, 1,
                              stacked=True, scalar_mode="cnt")
    a1e, scale = _prep2(S1, cntp.reshape(NP, 1), nw)
    T1, dnp = _sc_hconv_pass(a1e.reshape(2 * N, F), he3, 1, 0,
                             stacked=True, scalar_mode="dn", nw=nw[:, 0],
                             sched_dep=xij2)
    w_args = _light_wargs(graph_emb, Wc1, bc1, Wc2, bc2, attn)
    sij_main = _edge_light_main(xij2, bcol, w_args)

    Xl2, dinv = _prep3(T1, dnp.reshape(NP, 1), Whc2, bhc2)
    S2 = _sc_hconv_pass(Xl2, he3, 0, 1, stacked=False,
                        sched_dep=sij_main)
    a2 = _prep4(S2, scale)
    T2 = _sc_hconv_pass(a2, he3, 1, 0, stacked=False)

    sij_pref = _edge_light_prefix(xij2, bcol, T2, dinv, w_args)
    sij = jnp.concatenate([sij_pref, sij_main], axis=0)
    return (edge_index, edge_type, sij)
